# Initial kernel scaffold; baseline (speedup 1.0000x reference)
import jax
import jax.numpy as jnp
from jax.experimental import pallas as pl


def kernel(edge_index_1, features_1, batch_1, i_1, edge_index_2, features_2, batch_2, i_2, W1, b1, W2, b2, W3, b3, a1w1, a1b1, a1w2, a1b2, a2w1, a2b1, a2w2, a2b2, a3w1, a3b1, a3w2, a3b2, t1W, t1Wb, t1b, t2W, t2Wb, t2b, t3W, t3Wb, t3b, fc_w, fc_b, se_w1, se_b1, se_w2, se_b2):
    raise NotImplementedError("write your pallas kernel here")


# speedup vs baseline: 10.6143x; 10.6143x over previous
import functools

import jax
import jax.numpy as jnp
from jax import lax
from jax.experimental import pallas as pl
from jax.experimental.pallas import tpu as pltpu
from jax.experimental.pallas import tpu_sc as plsc

N = 10000
E = 320000
G = 100
F1, F2, F3 = 128, 64, 32

SC_CORES = 2
SC_TILES = 16
CH = 128
NCH = E // CH
NP = 10240
RPT = NP // SC_TILES

NB = 5
B = N // NB

_HI = jax.lax.Precision.HIGHEST


def _dot(a, b):
    return jnp.dot(a, b, precision=_HI, preferred_element_type=jnp.float32)


def _col0_to(width):
    r = lax.broadcasted_iota(jnp.int32, (16, width), 0)
    return (r == 0).astype(jnp.float32)


def _sc_mesh():
    return plsc.VectorSubcoreMesh(
        core_axis_name="c", subcore_axis_name="s",
        num_cores=SC_CORES, num_subcores=SC_TILES)


@functools.partial(jax.jit, static_argnames=("d",))
def _sc_edge_pass(hs_table, src_all, dst_all, zeros_tile, d):
    @functools.partial(
        pl.kernel,
        out_type=jax.ShapeDtypeStruct((SC_CORES, NP, d), jnp.float32),
        mesh=_sc_mesh(),
        compiler_params=pltpu.CompilerParams(use_tc_tiling_on_sc=False),
        scratch_types=[
            pltpu.VMEM((CH,), jnp.int32),
            pltpu.VMEM((CH,), jnp.int32),
            pltpu.VMEM((CH, d), jnp.float32),
            pltpu.VMEM_SHARED((NP, d), jnp.float32),
            pltpu.SemaphoreType.DMA,
        ],
    )
    def k(hs_hbm, src_hbm, dst_hbm, zero_hbm, out_hbm, src_v, dst_v, rows_v,
          acc_sh, sem):
        c = lax.axis_index("c")
        s = lax.axis_index("s")
        pltpu.sync_copy(zero_hbm, acc_sh.at[pl.ds(s * RPT, RPT)])
        plsc.subcore_barrier()

        nch_s = jnp.where(s < (NCH % SC_TILES), NCH // SC_TILES + 1,
                          NCH // SC_TILES).astype(jnp.int32)

        def body(t, carry):
            chunk_idx = s + t * SC_TILES
            off = c * E + chunk_idx * CH
            pltpu.sync_copy(src_hbm.at[pl.ds(off, CH)], src_v)
            pltpu.sync_copy(dst_hbm.at[pl.ds(off, CH)], dst_v)
            pltpu.async_copy(hs_hbm.at[src_v], rows_v, sem).wait()
            pltpu.sync_copy(rows_v, acc_sh.at[dst_v], add=True)
            return carry

        lax.fori_loop(0, nch_s, body, jnp.int32(0))
        plsc.subcore_barrier()
        pltpu.sync_copy(acc_sh.at[pl.ds(s * RPT, RPT)],
                        out_hbm.at[c, pl.ds(s * RPT, RPT)])

    return k(hs_table, src_all, dst_all, zeros_tile)


@jax.jit
def _sc_degree(dst_all, ones_rows, zeros_tile):
    @functools.partial(
        pl.kernel,
        out_type=jax.ShapeDtypeStruct((SC_CORES, NP, 16), jnp.float32),
        mesh=_sc_mesh(),
        compiler_params=pltpu.CompilerParams(use_tc_tiling_on_sc=False),
        scratch_types=[
            pltpu.VMEM((CH,), jnp.int32),
            pltpu.VMEM((CH, 16), jnp.float32),
            pltpu.VMEM_SHARED((NP, 16), jnp.float32),
        ],
    )
    def k(dst_hbm, ones_hbm, zero_hbm, out_hbm, dst_v, ones_v, acc_sh):
        c = lax.axis_index("c")
        s = lax.axis_index("s")
        pltpu.sync_copy(ones_hbm, ones_v)
        pltpu.sync_copy(zero_hbm, acc_sh.at[pl.ds(s * RPT, RPT)])
        plsc.subcore_barrier()

        nch_s = jnp.where(s < (NCH % SC_TILES), NCH // SC_TILES + 1,
                          NCH // SC_TILES).astype(jnp.int32)

        def body(t, carry):
            chunk_idx = s + t * SC_TILES
            off = c * E + chunk_idx * CH
            pltpu.sync_copy(dst_hbm.at[pl.ds(off, CH)], dst_v)
            pltpu.sync_copy(ones_v, acc_sh.at[dst_v], add=True)
            return carry

        lax.fori_loop(0, nch_s, body, jnp.int32(0))
        plsc.subcore_barrier()
        pltpu.sync_copy(acc_sh.at[pl.ds(s * RPT, RPT)],
                        out_hbm.at[c, pl.ds(s * RPT, RPT)])

    return k(dst_all, ones_rows, zeros_tile)


def _row_spec(d):
    return pl.BlockSpec((B, d), lambda i: (i, 0))


def _full_spec(shape):
    nd = len(shape)
    return pl.BlockSpec(shape, lambda i: (0,) * nd)


def _prep_body(degw_ref, x_ref, w1_ref, hs_ref):
    dinv_d = _dot(lax.rsqrt(degw_ref[...] + 1.0), _col0_to(F1))
    hs_ref[...] = dinv_d * _dot(x_ref[...], w1_ref[...])


@jax.jit
def _tc_prep(degw_g, x_g, W1):
    return pl.pallas_call(
        _prep_body,
        grid=(NB,),
        in_specs=[_row_spec(16), _row_spec(F1), _full_spec((F1, F1))],
        out_specs=_row_spec(F1),
        out_shape=jax.ShapeDtypeStruct((N, F1), jnp.float32),
    )(degw_g, x_g, W1)


def _layer_a_body(d, degw_ref, acc_ref, hs_ref, b_ref, br_ref,
                  aw1_ref, ab1_ref, aw2_ref, ab2_ref,
                  f_ref, sseg_ref, cnt_ref):
    dinv_d = _dot(lax.rsqrt(degw_ref[...] + 1.0), _col0_to(d))
    f = jnp.maximum(dinv_d * (acc_ref[...] + hs_ref[...]) + b_ref[...], 0.0)
    f_ref[...] = f
    u = jnp.maximum(_dot(f, aw1_ref[...]) + ab1_ref[...], 0.0)
    att = jnp.tanh(_dot(u, aw2_ref[...]) + ab2_ref[...])
    iota_gb = lax.broadcasted_iota(jnp.int32, (G, B), 0)
    br = br_ref[0]
    Pt = (jnp.broadcast_to(br, (G, B)) == iota_gb).astype(jnp.float32)
    sseg = _dot(Pt, f * att)
    cnt = _dot(Pt, jnp.ones((B, 1), jnp.float32))

    @pl.when(pl.program_id(0) == 0)
    def _():
        sseg_ref[...] = jnp.zeros_like(sseg_ref)
        cnt_ref[...] = jnp.zeros_like(cnt_ref)

    sseg_ref[...] += sseg
    cnt_ref[...] += cnt


@functools.partial(jax.jit, static_argnames=("d",))
def _tc_layer_a(degw_g, acc_g, hs_g, b, br3, aw1, ab1, aw2, ab2, d):
    r = aw1.shape[1]
    return pl.pallas_call(
        functools.partial(_layer_a_body, d),
        grid=(NB,),
        in_specs=[_row_spec(16), _row_spec(d), _row_spec(d),
                  _full_spec((1, d)),
                  pl.BlockSpec((1, 1, B), lambda i: (i, 0, 0)),
                  _full_spec((d, r)), _full_spec((1, r)),
                  _full_spec((r, d)), _full_spec((1, d))],
        out_specs=[_row_spec(d), _full_spec((G, d)), _full_spec((G, 1))],
        out_shape=[jax.ShapeDtypeStruct((N, d), jnp.float32),
                   jax.ShapeDtypeStruct((G, d), jnp.float32),
                   jax.ShapeDtypeStruct((G, 1), jnp.float32)],
    )(degw_g, acc_g, hs_g, b, br3, aw1, ab1, aw2, ab2)


def _layer_b_body(d, dnext, degw_ref, f_ref, bc_ref, br_ref, sseg_ref,
                  cnt_ref, wn_ref, e_ref, hsn_ref):
    f = f_ref[...]
    tg = jnp.tanh(sseg_ref[...] / jnp.maximum(cnt_ref[...], 1.0))
    bc = bc_ref[0]
    br = br_ref[0]
    iota_bg = lax.broadcasted_iota(jnp.int32, (B, G), 1)
    iota_gb = lax.broadcasted_iota(jnp.int32, (G, B), 0)
    P = (jnp.broadcast_to(bc, (B, G)) == iota_bg).astype(jnp.float32)
    Pt = (jnp.broadcast_to(br, (G, B)) == iota_gb).astype(jnp.float32)
    tgn = _dot(P, tg)
    coefs_d = jax.nn.sigmoid(
        _dot(f * tgn, jnp.ones((d, d), jnp.float32)) * 10.0)
    e = _dot(Pt, coefs_d * f)

    @pl.when(pl.program_id(0) == 0)
    def _():
        e_ref[...] = jnp.zeros_like(e_ref)

    e_ref[...] += e
    if dnext:
        dinv_dn = _dot(lax.rsqrt(degw_ref[...] + 1.0), _col0_to(dnext))
        hsn_ref[...] = dinv_dn * _dot(f, wn_ref[...])


@functools.partial(jax.jit, static_argnames=("d", "dnext"))
def _tc_layer_b(degw_g, f, bc3, br3, sseg, cnt, Wnext, d, dnext):
    dn = dnext or 8
    out_specs = [_full_spec((G, d)), _row_spec(dn)]
    out_shape = [jax.ShapeDtypeStruct((G, d), jnp.float32),
                 jax.ShapeDtypeStruct((N, dn), jnp.float32)]
    return pl.pallas_call(
        functools.partial(_layer_b_body, d, dnext),
        grid=(NB,),
        in_specs=[_row_spec(16), _row_spec(d),
                  pl.BlockSpec((1, B, 1), lambda i: (i, 0, 0)),
                  pl.BlockSpec((1, 1, B), lambda i: (i, 0, 0)),
                  _full_spec((G, d)), _full_spec((G, 1)),
                  _full_spec((d, dn))],
        out_specs=out_specs,
        out_shape=out_shape,
    )(degw_g, f, bc3, br3, sseg, cnt, Wnext)


def _ntn_body(d, dh, e1_ref, e2_ref, tW_ref, tWbT_ref, tb_ref, s_ref):
    T3 = _dot(e1_ref[...], tW_ref[...])
    scoring = jnp.sum(T3.reshape(G, d, dh) * e2_ref[...][:, :, None], axis=1)
    block = _dot(jnp.concatenate([e1_ref[...], e2_ref[...]], axis=1),
                 tWbT_ref[...])
    s_ref[...] = jnp.maximum(scoring + block + tb_ref[...], 0.0)


@functools.partial(jax.jit, static_argnames=("d", "dh"))
def _tc_ntn(e1, e2, tWf, tWbT, tb, d, dh):
    body = functools.partial(_ntn_body, d, dh)
    return pl.pallas_call(
        body, out_shape=jax.ShapeDtypeStruct((G, dh), jnp.float32))(
        e1, e2, tWf, tWbT, tb)


def _head_body(s3_ref, s2_ref, s1_ref, sew1_ref, seb1_ref, sew2_ref,
               seb2_ref, fcw_ref, fcb_ref, out_ref):
    scores = jnp.concatenate([s3_ref[...], s2_ref[...], s1_ref[...]], axis=1)
    se = jax.nn.sigmoid(
        _dot(jnp.maximum(_dot(scores, sew1_ref[...]) + seb1_ref[...], 0.0),
             sew2_ref[...]) + seb2_ref[...])
    out_ref[...] = jnp.maximum(
        _dot(se * scores + scores, fcw_ref[...]) + fcb_ref[...], 0.0)


@jax.jit
def _tc_head(s3, s2, s1, sew1, seb1, sew2, seb2, fcw, fcb):
    return pl.pallas_call(
        _head_body, out_shape=jax.ShapeDtypeStruct((G, 64), jnp.float32))(
        s3, s2, s1, sew1, seb1, sew2, seb2, fcw, fcb)


def kernel(edge_index_1, features_1, batch_1, i_1, edge_index_2, features_2,
           batch_2, i_2, W1, b1, W2, b2, W3, b3, a1w1, a1b1, a1w2, a1b2,
           a2w1, a2b1, a2w2, a2b2, a3w1, a3b1, a3w2, a3b2, t1W, t1Wb, t1b,
           t2W, t2Wb, t2b, t3W, t3Wb, t3b, fc_w, fc_b, se_w1, se_b1,
           se_w2, se_b2):
    src_all = jnp.concatenate([edge_index_1[0], edge_index_2[0] + N])
    dst_all = jnp.concatenate([edge_index_1[1], edge_index_2[1]])
    bc3 = (batch_1.reshape(NB, B, 1), batch_2.reshape(NB, B, 1))
    br3 = (batch_1.reshape(NB, 1, B), batch_2.reshape(NB, 1, B))
    tWf = (t1W.reshape(F1, F1 * (F1 // 2)),
           t2W.reshape(F2, F2 * (F2 // 2)),
           t3W.reshape(F3, F3 * (F3 // 2)))
    tWbT = (t1Wb.T, t2Wb.T, t3Wb.T)
    tb = (t1b.reshape(1, -1), t2b.reshape(1, -1), t3b.reshape(1, -1))
    aws = ((a1w1, a1b1.reshape(1, -1), a1w2, a1b2.reshape(1, -1)),
           (a2w1, a2b1.reshape(1, -1), a2w2, a2b2.reshape(1, -1)),
           (a3w1, a3b1.reshape(1, -1), a3w2, a3b2.reshape(1, -1)))
    bs = (b1.reshape(1, F1), b2.reshape(1, F2), b3.reshape(1, F3))
    Wn = (W2, W3, None)
    dims = (F1, F2, F3)
    ones_rows = jnp.ones((CH, 16), jnp.float32)
    z16 = jnp.zeros((RPT, 16), jnp.float32)
    zd = {dd: jnp.zeros((RPT, dd), jnp.float32) for dd in dims}

    degw = _sc_degree(dst_all, ones_rows, z16)
    degw_g = (degw[0, :N], degw[1, :N])
    hs = [_tc_prep(degw_g[0], features_1, W1),
          _tc_prep(degw_g[1], features_2, W1)]

    ss = []
    for l in range(3):
        d, dnext = dims[l], (dims[l + 1] if l < 2 else 0)
        hs_table = jnp.concatenate(hs, axis=0)
        acc = _sc_edge_pass(hs_table, src_all, dst_all, zd[d], d=d)
        es = []
        for g in range(2):
            aw1, ab1, aw2, ab2 = aws[l]
            f, sseg, cnt = _tc_layer_a(degw_g[g], acc[g, :N], hs[g], bs[l],
                                       br3[g], aw1, ab1, aw2, ab2, d=d)
            wn = Wn[l] if Wn[l] is not None else jnp.zeros((d, 8), jnp.float32)
            e_g, hsn_g = _tc_layer_b(degw_g[g], f, bc3[g], br3[g], sseg, cnt,
                                     wn, d=d, dnext=dnext)
            es.append(e_g)
            hs[g] = hsn_g
        ss.append(_tc_ntn(es[0], es[1], tWf[l], tWbT[l], tb[l],
                          d=d, dh=d // 2))

    return _tc_head(ss[2], ss[1], ss[0], se_w1, se_b1.reshape(1, -1),
                    se_w2, se_b2.reshape(1, -1), fc_w, fc_b.reshape(1, -1))


# speedup vs baseline: 18.9283x; 1.7833x over previous
import functools

import jax
import jax.numpy as jnp
from jax import lax
from jax.experimental import pallas as pl
from jax.experimental.pallas import tpu as pltpu
from jax.experimental.pallas import tpu_sc as plsc

N = 10000
E = 320000
G = 100
F1, F2, F3 = 128, 64, 32

SC_CORES = 2
SC_TILES = 16
EPT = E // SC_TILES
ECH = 125
TCH = EPT // ECH
NP = 10240
RPT = NP // SC_TILES

NB = 5
B = N // NB

_HI = jax.lax.Precision.HIGHEST


def _dot(a, b):
    return jnp.dot(a, b, precision=_HI, preferred_element_type=jnp.float32)


def _col0_to(width):
    r = lax.broadcasted_iota(jnp.int32, (16, width), 0)
    return (r == 0).astype(jnp.float32)


def _sc_mesh():
    return plsc.VectorSubcoreMesh(
        core_axis_name="c", subcore_axis_name="s",
        num_cores=SC_CORES, num_subcores=SC_TILES)


@functools.partial(jax.jit, static_argnames=("d", "kdepth"))
def _sc_edge_pass(hs_table, src_flat, dst_flat, zeros_tile, d, kdepth=1):
    K = kdepth
    NSLOT = 2 * K
    BCH = 16
    NBLK = TCH // BCH
    assert TCH % NSLOT == 0 and BCH % NSLOT == 0 and BCH > 2 * K
    src_r = src_flat.reshape(SC_CORES * SC_TILES, NBLK, BCH, ECH)
    dst_r = dst_flat.reshape(SC_CORES * SC_TILES, NBLK, BCH, ECH)

    @functools.partial(
        pl.kernel,
        out_type=jax.ShapeDtypeStruct((SC_CORES, NP, d), jnp.float32),
        mesh=_sc_mesh(),
        compiler_params=pltpu.CompilerParams(use_tc_tiling_on_sc=False),
        scratch_types=[
            pltpu.VMEM((2, BCH, ECH), jnp.int32),
            pltpu.VMEM((2, BCH, ECH), jnp.int32),
            pltpu.VMEM((NSLOT, ECH, d), jnp.float32),
            pltpu.VMEM_SHARED((NP, d), jnp.float32),
        ] + [pltpu.SemaphoreType.DMA] * (2 * NSLOT + 1),
    )
    def k(hs_hbm, src_hbm, dst_hbm, zero_hbm, out_hbm, src_v, dst_v, rows_v,
          acc_sh, *sems):
        gsem = sems[:NSLOT]
        ssem = sems[NSLOT:2 * NSLOT]
        isem = sems[2 * NSLOT]
        c = lax.axis_index("c")
        s = lax.axis_index("s")
        w = c * SC_TILES + s
        pltpu.sync_copy(zero_hbm, acc_sh.at[pl.ds(s * RPT, RPT)])
        pltpu.sync_copy(src_hbm.at[w, 0], src_v.at[0])
        pltpu.sync_copy(dst_hbm.at[w, 0], dst_v.at[0])
        plsc.subcore_barrier()

        for j in range(K):
            pltpu.async_copy(hs_hbm.at[src_v.at[0].at[j]], rows_v.at[j],
                             gsem[j])

        def body(tt, carry):
            for j in range(NSLOT):
                t = tt * NSLOT + j
                jk = (j + K) % NSLOT
                bb = t // BCH
                r = t % BCH
                p = bb % 2
                t2 = t + K
                p2 = (t2 // BCH) % 2
                r2 = t2 % BCH

                @pl.when((r == K) & (bb + 1 < NBLK))
                def _():
                    pn = (bb + 1) % 2
                    pltpu.async_copy(src_hbm.at[w, bb + 1], src_v.at[pn],
                                     isem)
                    pltpu.async_copy(dst_hbm.at[w, bb + 1], dst_v.at[pn],
                                     isem)

                @pl.when((r == BCH - K) & (bb + 1 < NBLK))
                def _():
                    pltpu.make_async_copy(src_hbm.at[w, 0], src_v.at[0],
                                          isem).wait()
                    pltpu.make_async_copy(dst_hbm.at[w, 0], dst_v.at[0],
                                          isem).wait()

                pltpu.make_async_copy(
                    hs_hbm.at[src_v.at[p].at[r]], rows_v.at[j],
                    gsem[j]).wait()
                pltpu.async_copy(rows_v.at[j], acc_sh.at[dst_v.at[p].at[r]],
                                 ssem[j], add=True)

                @pl.when(t >= K)
                def _():
                    pltpu.make_async_copy(
                        rows_v.at[jk], acc_sh.at[dst_v.at[p].at[r]],
                        ssem[jk]).wait()

                @pl.when(t2 < TCH)
                def _():
                    pltpu.async_copy(hs_hbm.at[src_v.at[p2].at[r2]],
                                     rows_v.at[jk], gsem[jk])
            return carry

        lax.fori_loop(0, TCH // NSLOT, body, jnp.int32(0))
        for j in range(K, NSLOT):
            pltpu.make_async_copy(
                rows_v.at[j], acc_sh.at[dst_v.at[0].at[0]], ssem[j]).wait()
        plsc.subcore_barrier()
        pltpu.sync_copy(acc_sh.at[pl.ds(s * RPT, RPT)],
                        out_hbm.at[c, pl.ds(s * RPT, RPT)])

    return k(hs_table, src_r, dst_r, zeros_tile)


@jax.jit
def _sc_degree(dst_flat, ones_rows, zeros_tile):
    dst_r = dst_flat.reshape(SC_CORES * SC_TILES, TCH, ECH)

    @functools.partial(
        pl.kernel,
        out_type=jax.ShapeDtypeStruct((SC_CORES, NP, 16), jnp.float32),
        mesh=_sc_mesh(),
        compiler_params=pltpu.CompilerParams(use_tc_tiling_on_sc=False),
        scratch_types=[
            pltpu.VMEM((TCH, ECH), jnp.int32),
            pltpu.VMEM((ECH, 16), jnp.float32),
            pltpu.VMEM_SHARED((NP, 16), jnp.float32),
            pltpu.SemaphoreType.DMA,
        ],
    )
    def k(dst_hbm, ones_hbm, zero_hbm, out_hbm, dst_v, ones_v, acc_sh, ssem):
        c = lax.axis_index("c")
        s = lax.axis_index("s")
        w = c * SC_TILES + s
        pltpu.sync_copy(ones_hbm, ones_v)
        pltpu.sync_copy(dst_hbm.at[w], dst_v)
        pltpu.sync_copy(zero_hbm, acc_sh.at[pl.ds(s * RPT, RPT)])
        plsc.subcore_barrier()

        def body(t, carry):
            pltpu.async_copy(ones_v, acc_sh.at[dst_v.at[t]], ssem, add=True)
            return carry

        lax.fori_loop(0, TCH, body, jnp.int32(0))

        def drain(t, carry):
            pltpu.make_async_copy(ones_v, acc_sh.at[dst_v.at[0]],
                                  ssem).wait()
            return carry

        lax.fori_loop(0, TCH, drain, jnp.int32(0))
        plsc.subcore_barrier()
        pltpu.sync_copy(acc_sh.at[pl.ds(s * RPT, RPT)],
                        out_hbm.at[c, pl.ds(s * RPT, RPT)])

    return k(dst_r, ones_rows, zeros_tile)


def _row_spec(d):
    return pl.BlockSpec((B, d), lambda i: (i, 0))


def _full_spec(shape):
    nd = len(shape)
    return pl.BlockSpec(shape, lambda i: (0,) * nd)


def _prep_body(degw_ref, x_ref, w1_ref, hs_ref):
    dinv_d = _dot(lax.rsqrt(degw_ref[...] + 1.0), _col0_to(F1))
    hs_ref[...] = dinv_d * _dot(x_ref[...], w1_ref[...])


@jax.jit
def _tc_prep(degw_g, x_g, W1):
    return pl.pallas_call(
        _prep_body,
        grid=(NB,),
        in_specs=[_row_spec(16), _row_spec(F1), _full_spec((F1, F1))],
        out_specs=_row_spec(F1),
        out_shape=jax.ShapeDtypeStruct((N, F1), jnp.float32),
    )(degw_g, x_g, W1)


def _layer_a_body(d, degw_ref, acc_ref, hs_ref, b_ref, br_ref,
                  aw1_ref, ab1_ref, aw2_ref, ab2_ref,
                  f_ref, sseg_ref, cnt_ref):
    dinv_d = _dot(lax.rsqrt(degw_ref[...] + 1.0), _col0_to(d))
    f = jnp.maximum(dinv_d * (acc_ref[...] + hs_ref[...]) + b_ref[...], 0.0)
    f_ref[...] = f
    u = jnp.maximum(_dot(f, aw1_ref[...]) + ab1_ref[...], 0.0)
    att = jnp.tanh(_dot(u, aw2_ref[...]) + ab2_ref[...])
    iota_gb = lax.broadcasted_iota(jnp.int32, (G, B), 0)
    br = br_ref[0]
    Pt = (jnp.broadcast_to(br, (G, B)) == iota_gb).astype(jnp.float32)
    sseg = _dot(Pt, f * att)
    cnt = _dot(Pt, jnp.ones((B, 1), jnp.float32))

    @pl.when(pl.program_id(0) == 0)
    def _():
        sseg_ref[...] = jnp.zeros_like(sseg_ref)
        cnt_ref[...] = jnp.zeros_like(cnt_ref)

    sseg_ref[...] += sseg
    cnt_ref[...] += cnt


@functools.partial(jax.jit, static_argnames=("d",))
def _tc_layer_a(degw_g, acc_g, hs_g, b, br3, aw1, ab1, aw2, ab2, d):
    r = aw1.shape[1]
    return pl.pallas_call(
        functools.partial(_layer_a_body, d),
        grid=(NB,),
        in_specs=[_row_spec(16), _row_spec(d), _row_spec(d),
                  _full_spec((1, d)),
                  pl.BlockSpec((1, 1, B), lambda i: (i, 0, 0)),
                  _full_spec((d, r)), _full_spec((1, r)),
                  _full_spec((r, d)), _full_spec((1, d))],
        out_specs=[_row_spec(d), _full_spec((G, d)), _full_spec((G, 1))],
        out_shape=[jax.ShapeDtypeStruct((N, d), jnp.float32),
                   jax.ShapeDtypeStruct((G, d), jnp.float32),
                   jax.ShapeDtypeStruct((G, 1), jnp.float32)],
    )(degw_g, acc_g, hs_g, b, br3, aw1, ab1, aw2, ab2)


def _layer_b_body(d, dnext, degw_ref, f_ref, bc_ref, br_ref, sseg_ref,
                  cnt_ref, wn_ref, e_ref, hsn_ref):
    f = f_ref[...]
    tg = jnp.tanh(sseg_ref[...] / jnp.maximum(cnt_ref[...], 1.0))
    bc = bc_ref[0]
    br = br_ref[0]
    iota_bg = lax.broadcasted_iota(jnp.int32, (B, G), 1)
    iota_gb = lax.broadcasted_iota(jnp.int32, (G, B), 0)
    P = (jnp.broadcast_to(bc, (B, G)) == iota_bg).astype(jnp.float32)
    Pt = (jnp.broadcast_to(br, (G, B)) == iota_gb).astype(jnp.float32)
    tgn = _dot(P, tg)
    coefs_d = jax.nn.sigmoid(
        _dot(f * tgn, jnp.ones((d, d), jnp.float32)) * 10.0)
    e = _dot(Pt, coefs_d * f)

    @pl.when(pl.program_id(0) == 0)
    def _():
        e_ref[...] = jnp.zeros_like(e_ref)

    e_ref[...] += e
    if dnext:
        dinv_dn = _dot(lax.rsqrt(degw_ref[...] + 1.0), _col0_to(dnext))
        hsn_ref[...] = dinv_dn * _dot(f, wn_ref[...])


@functools.partial(jax.jit, static_argnames=("d", "dnext"))
def _tc_layer_b(degw_g, f, bc3, br3, sseg, cnt, Wnext, d, dnext):
    dn = dnext or 8
    out_specs = [_full_spec((G, d)), _row_spec(dn)]
    out_shape = [jax.ShapeDtypeStruct((G, d), jnp.float32),
                 jax.ShapeDtypeStruct((N, dn), jnp.float32)]
    return pl.pallas_call(
        functools.partial(_layer_b_body, d, dnext),
        grid=(NB,),
        in_specs=[_row_spec(16), _row_spec(d),
                  pl.BlockSpec((1, B, 1), lambda i: (i, 0, 0)),
                  pl.BlockSpec((1, 1, B), lambda i: (i, 0, 0)),
                  _full_spec((G, d)), _full_spec((G, 1)),
                  _full_spec((d, dn))],
        out_specs=out_specs,
        out_shape=out_shape,
    )(degw_g, f, bc3, br3, sseg, cnt, Wnext)


def _ntn_body(d, dh, e1_ref, e2_ref, tW_ref, tWbT_ref, tb_ref, s_ref):
    T3 = _dot(e1_ref[...], tW_ref[...])
    scoring = jnp.sum(T3.reshape(G, d, dh) * e2_ref[...][:, :, None], axis=1)
    block = _dot(jnp.concatenate([e1_ref[...], e2_ref[...]], axis=1),
                 tWbT_ref[...])
    s_ref[...] = jnp.maximum(scoring + block + tb_ref[...], 0.0)


@functools.partial(jax.jit, static_argnames=("d", "dh"))
def _tc_ntn(e1, e2, tWf, tWbT, tb, d, dh):
    body = functools.partial(_ntn_body, d, dh)
    return pl.pallas_call(
        body, out_shape=jax.ShapeDtypeStruct((G, dh), jnp.float32))(
        e1, e2, tWf, tWbT, tb)


def _head_body(s3_ref, s2_ref, s1_ref, sew1_ref, seb1_ref, sew2_ref,
               seb2_ref, fcw_ref, fcb_ref, out_ref):
    scores = jnp.concatenate([s3_ref[...], s2_ref[...], s1_ref[...]], axis=1)
    se = jax.nn.sigmoid(
        _dot(jnp.maximum(_dot(scores, sew1_ref[...]) + seb1_ref[...], 0.0),
             sew2_ref[...]) + seb2_ref[...])
    out_ref[...] = jnp.maximum(
        _dot(se * scores + scores, fcw_ref[...]) + fcb_ref[...], 0.0)


@jax.jit
def _tc_head(s3, s2, s1, sew1, seb1, sew2, seb2, fcw, fcb):
    return pl.pallas_call(
        _head_body, out_shape=jax.ShapeDtypeStruct((G, 64), jnp.float32))(
        s3, s2, s1, sew1, seb1, sew2, seb2, fcw, fcb)


def kernel(edge_index_1, features_1, batch_1, i_1, edge_index_2, features_2,
           batch_2, i_2, W1, b1, W2, b2, W3, b3, a1w1, a1b1, a1w2, a1b2,
           a2w1, a2b1, a2w2, a2b2, a3w1, a3b1, a3w2, a3b2, t1W, t1Wb, t1b,
           t2W, t2Wb, t2b, t3W, t3Wb, t3b, fc_w, fc_b, se_w1, se_b1,
           se_w2, se_b2):
    src_flat = jnp.concatenate([edge_index_1[0], edge_index_2[0] + N])
    dst_flat = jnp.concatenate([edge_index_1[1], edge_index_2[1]])
    bc3 = (batch_1.reshape(NB, B, 1), batch_2.reshape(NB, B, 1))
    br3 = (batch_1.reshape(NB, 1, B), batch_2.reshape(NB, 1, B))
    tWf = (t1W.reshape(F1, F1 * (F1 // 2)),
           t2W.reshape(F2, F2 * (F2 // 2)),
           t3W.reshape(F3, F3 * (F3 // 2)))
    tWbT = (t1Wb.T, t2Wb.T, t3Wb.T)
    tb = (t1b.reshape(1, -1), t2b.reshape(1, -1), t3b.reshape(1, -1))
    aws = ((a1w1, a1b1.reshape(1, -1), a1w2, a1b2.reshape(1, -1)),
           (a2w1, a2b1.reshape(1, -1), a2w2, a2b2.reshape(1, -1)),
           (a3w1, a3b1.reshape(1, -1), a3w2, a3b2.reshape(1, -1)))
    bs = (b1.reshape(1, F1), b2.reshape(1, F2), b3.reshape(1, F3))
    Wn = (W2, W3, None)
    dims = (F1, F2, F3)
    ones_rows = jnp.ones((ECH, 16), jnp.float32)
    z16 = jnp.zeros((RPT, 16), jnp.float32)
    zd = {dd: jnp.zeros((RPT, dd), jnp.float32) for dd in dims}

    degw = _sc_degree(dst_flat, ones_rows, z16)
    degw_g = (degw[0, :N], degw[1, :N])
    hs = [_tc_prep(degw_g[0], features_1, W1),
          _tc_prep(degw_g[1], features_2, W1)]

    ss = []
    for l in range(3):
        d, dnext = dims[l], (dims[l + 1] if l < 2 else 0)
        hs_table = jnp.concatenate(hs, axis=0)
        acc = _sc_edge_pass(hs_table, src_flat, dst_flat, zd[d], d=d,
                            kdepth=1 if d == F1 else 4)
        es = []
        for g in range(2):
            aw1, ab1, aw2, ab2 = aws[l]
            f, sseg, cnt = _tc_layer_a(degw_g[g], acc[g, :N], hs[g], bs[l],
                                       br3[g], aw1, ab1, aw2, ab2, d=d)
            wn = Wn[l] if Wn[l] is not None else jnp.zeros((d, 8), jnp.float32)
            e_g, hsn_g = _tc_layer_b(degw_g[g], f, bc3[g], br3[g], sseg, cnt,
                                     wn, d=d, dnext=dnext)
            es.append(e_g)
            hs[g] = hsn_g
        ss.append(_tc_ntn(es[0], es[1], tWf[l], tWbT[l], tb[l],
                          d=d, dh=d // 2))

    return _tc_head(ss[2], ss[1], ss[0], se_w1, se_b1.reshape(1, -1),
                    se_w2, se_b2.reshape(1, -1), fc_w, fc_b.reshape(1, -1))


# speedup vs baseline: 20.9285x; 1.1057x over previous
import functools

import jax
import jax.numpy as jnp
from jax import lax
from jax.experimental import pallas as pl
from jax.experimental.pallas import tpu as pltpu
from jax.experimental.pallas import tpu_sc as plsc

N = 10000
E = 320000
G = 100
F1, F2, F3 = 128, 64, 32

SC_CORES = 2
SC_TILES = 16
EPT = E // SC_TILES
ECH = 125
TCH = EPT // ECH
NP = 10240
RPT = NP // SC_TILES

NB = 5
B = N // NB

_HI = jax.lax.Precision.HIGHEST


def _dot(a, b):
    return jnp.dot(a, b, precision=_HI, preferred_element_type=jnp.float32)


def _col0_to(width):
    r = lax.broadcasted_iota(jnp.int32, (16, width), 0)
    return (r == 0).astype(jnp.float32)


def _sc_mesh():
    return plsc.VectorSubcoreMesh(
        core_axis_name="c", subcore_axis_name="s",
        num_cores=SC_CORES, num_subcores=SC_TILES)


@functools.partial(jax.jit, static_argnames=("d", "kdepth"))
def _sc_edge_pass(hs_table, src_flat, dst_flat, zeros_tile, d, kdepth=1):
    K = kdepth
    NSLOT = 2 * K
    BCH = 16
    NBLK = TCH // BCH
    assert TCH % NSLOT == 0 and BCH % NSLOT == 0 and BCH > 2 * K
    src_r = src_flat.reshape(SC_CORES * SC_TILES, NBLK, BCH, ECH)
    dst_r = dst_flat.reshape(SC_CORES * SC_TILES, NBLK, BCH, ECH)

    @functools.partial(
        pl.kernel,
        out_type=jax.ShapeDtypeStruct((SC_CORES, NP, d), jnp.float32),
        mesh=_sc_mesh(),
        compiler_params=pltpu.CompilerParams(use_tc_tiling_on_sc=False),
        scratch_types=[
            pltpu.VMEM((2, BCH, ECH), jnp.int32),
            pltpu.VMEM((2, BCH, ECH), jnp.int32),
            pltpu.VMEM((NSLOT, ECH, d), jnp.float32),
            pltpu.VMEM_SHARED((NP, d), jnp.float32),
        ] + [pltpu.SemaphoreType.DMA] * (2 * NSLOT + 1),
    )
    def k(hs_hbm, src_hbm, dst_hbm, zero_hbm, out_hbm, src_v, dst_v, rows_v,
          acc_sh, *sems):
        gsem = sems[:NSLOT]
        ssem = sems[NSLOT:2 * NSLOT]
        isem = sems[2 * NSLOT]
        c = lax.axis_index("c")
        s = lax.axis_index("s")
        w = c * SC_TILES + s
        pltpu.sync_copy(zero_hbm, acc_sh.at[pl.ds(s * RPT, RPT)])
        pltpu.sync_copy(src_hbm.at[w, 0], src_v.at[0])
        pltpu.sync_copy(dst_hbm.at[w, 0], dst_v.at[0])
        plsc.subcore_barrier()

        for j in range(K):
            pltpu.async_copy(hs_hbm.at[src_v.at[0].at[j]], rows_v.at[j],
                             gsem[j])

        def body(tt, carry):
            for j in range(NSLOT):
                t = tt * NSLOT + j
                jk = (j + K) % NSLOT
                bb = t // BCH
                r = t % BCH
                p = bb % 2
                t2 = t + K
                p2 = (t2 // BCH) % 2
                r2 = t2 % BCH

                @pl.when((r == K) & (bb + 1 < NBLK))
                def _():
                    pn = (bb + 1) % 2
                    pltpu.async_copy(src_hbm.at[w, bb + 1], src_v.at[pn],
                                     isem)
                    pltpu.async_copy(dst_hbm.at[w, bb + 1], dst_v.at[pn],
                                     isem)

                @pl.when((r == BCH - K) & (bb + 1 < NBLK))
                def _():
                    pltpu.make_async_copy(src_hbm.at[w, 0], src_v.at[0],
                                          isem).wait()
                    pltpu.make_async_copy(dst_hbm.at[w, 0], dst_v.at[0],
                                          isem).wait()

                pltpu.make_async_copy(
                    hs_hbm.at[src_v.at[p].at[r]], rows_v.at[j],
                    gsem[j]).wait()
                pltpu.async_copy(rows_v.at[j], acc_sh.at[dst_v.at[p].at[r]],
                                 ssem[j], add=True)

                @pl.when(t >= K)
                def _():
                    pltpu.make_async_copy(
                        rows_v.at[jk], acc_sh.at[dst_v.at[p].at[r]],
                        ssem[jk]).wait()

                @pl.when(t2 < TCH)
                def _():
                    pltpu.async_copy(hs_hbm.at[src_v.at[p2].at[r2]],
                                     rows_v.at[jk], gsem[jk])
            return carry

        lax.fori_loop(0, TCH // NSLOT, body, jnp.int32(0))
        for j in range(K, NSLOT):
            pltpu.make_async_copy(
                rows_v.at[j], acc_sh.at[dst_v.at[0].at[0]], ssem[j]).wait()
        plsc.subcore_barrier()
        pltpu.sync_copy(acc_sh.at[pl.ds(s * RPT, RPT)],
                        out_hbm.at[c, pl.ds(s * RPT, RPT)])

    return k(hs_table, src_r, dst_r, zeros_tile)


@jax.jit
def _sc_degree(dst_flat, ones_rows, zeros_tile):
    dst_r = dst_flat.reshape(SC_CORES * SC_TILES, TCH, ECH)

    @functools.partial(
        pl.kernel,
        out_type=jax.ShapeDtypeStruct((SC_CORES, NP, 16), jnp.float32),
        mesh=_sc_mesh(),
        compiler_params=pltpu.CompilerParams(use_tc_tiling_on_sc=False),
        scratch_types=[
            pltpu.VMEM((TCH, ECH), jnp.int32),
            pltpu.VMEM((ECH, 16), jnp.float32),
            pltpu.VMEM_SHARED((NP, 16), jnp.float32),
            pltpu.SemaphoreType.DMA,
        ],
    )
    def k(dst_hbm, ones_hbm, zero_hbm, out_hbm, dst_v, ones_v, acc_sh, ssem):
        c = lax.axis_index("c")
        s = lax.axis_index("s")
        w = c * SC_TILES + s
        pltpu.sync_copy(ones_hbm, ones_v)
        pltpu.sync_copy(dst_hbm.at[w], dst_v)
        pltpu.sync_copy(zero_hbm, acc_sh.at[pl.ds(s * RPT, RPT)])
        plsc.subcore_barrier()

        def body(t, carry):
            pltpu.async_copy(ones_v, acc_sh.at[dst_v.at[t]], ssem, add=True)
            return carry

        lax.fori_loop(0, TCH, body, jnp.int32(0))

        def drain(t, carry):
            pltpu.make_async_copy(ones_v, acc_sh.at[dst_v.at[0]],
                                  ssem).wait()
            return carry

        lax.fori_loop(0, TCH, drain, jnp.int32(0))
        plsc.subcore_barrier()
        pltpu.sync_copy(acc_sh.at[pl.ds(s * RPT, RPT)],
                        out_hbm.at[c, pl.ds(s * RPT, RPT)])

    return k(dst_r, ones_rows, zeros_tile)


def _row2_spec(d):
    return pl.BlockSpec((2, B, d), lambda i: (0, i, 0))


def _full_spec(shape):
    nd = len(shape)
    return pl.BlockSpec(shape, lambda i: (0,) * nd)


def _prep_body(degw_ref, x1_ref, x2_ref, w1_ref, hs_ref):
    for g, x_ref in ((0, x1_ref), (1, x2_ref)):
        dinv_d = _dot(lax.rsqrt(degw_ref[g] + 1.0), _col0_to(F1))
        hs_ref[g] = dinv_d * _dot(x_ref[...], w1_ref[...])


@jax.jit
def _tc_prep(degw, x1, x2, W1):
    return pl.pallas_call(
        _prep_body,
        grid=(NB,),
        in_specs=[pl.BlockSpec((2, B, 16), lambda i: (0, i, 0)),
                  pl.BlockSpec((B, F1), lambda i: (i, 0)),
                  pl.BlockSpec((B, F1), lambda i: (i, 0)),
                  _full_spec((F1, F1))],
        out_specs=_row2_spec(F1),
        out_shape=jax.ShapeDtypeStruct((2, N, F1), jnp.float32),
    )(degw, x1, x2, W1)


def _layer_a_body(d, degw_ref, acc_ref, hs_ref, b_ref, br_ref,
                  aw1_ref, ab1_ref, aw2_ref, ab2_ref,
                  f_ref, sseg_ref, cnt_ref):
    i = pl.program_id(0)
    iota_gb = lax.broadcasted_iota(jnp.int32, (G, B), 0)

    @pl.when(i == 0)
    def _():
        sseg_ref[...] = jnp.zeros_like(sseg_ref)
        cnt_ref[...] = jnp.zeros_like(cnt_ref)

    for g in range(2):
        dinv_d = _dot(lax.rsqrt(degw_ref[g] + 1.0), _col0_to(d))
        f = jnp.maximum(dinv_d * (acc_ref[g] + hs_ref[g]) + b_ref[...], 0.0)
        f_ref[g] = f
        u = jnp.maximum(_dot(f, aw1_ref[...]) + ab1_ref[...], 0.0)
        att = jnp.tanh(_dot(u, aw2_ref[...]) + ab2_ref[...])
        Pt = (jnp.broadcast_to(br_ref[g, 0], (G, B)) == iota_gb
              ).astype(jnp.float32)
        sseg_ref[g] += _dot(Pt, f * att)
        cnt_ref[g] += _dot(Pt, jnp.ones((B, 1), jnp.float32))


@functools.partial(jax.jit, static_argnames=("d",))
def _tc_layer_a(degw, acc, hs, b, br3, aw1, ab1, aw2, ab2, d):
    r = aw1.shape[1]
    return pl.pallas_call(
        functools.partial(_layer_a_body, d),
        grid=(NB,),
        in_specs=[pl.BlockSpec((2, B, 16), lambda i: (0, i, 0)),
                  _row2_spec(d), _row2_spec(d),
                  _full_spec((1, d)),
                  pl.BlockSpec((2, 1, 1, B), lambda i: (0, i, 0, 0)),
                  _full_spec((d, r)), _full_spec((1, r)),
                  _full_spec((r, d)), _full_spec((1, d))],
        out_specs=[_row2_spec(d), _full_spec((2, G, d)),
                   _full_spec((2, G, 1))],
        out_shape=[jax.ShapeDtypeStruct((2, N, d), jnp.float32),
                   jax.ShapeDtypeStruct((2, G, d), jnp.float32),
                   jax.ShapeDtypeStruct((2, G, 1), jnp.float32)],
    )(degw, acc, hs, b, br3, aw1, ab1, aw2, ab2)


def _ntn(e1, e2, tW, tWbT, tb, d, dh):
    T3 = _dot(e1, tW)
    scoring = jnp.sum(T3.reshape(G, d, dh) * e2[:, :, None], axis=1)
    block = _dot(jnp.concatenate([e1, e2], axis=1), tWbT)
    return jnp.maximum(scoring + block + tb, 0.0)


def _layer_b_body(d, dnext, final, *refs):
    if final:
        (degw_ref, f_ref, bc_ref, br_ref, sseg_ref, cnt_ref, wn_ref,
         tW_ref, tWbT_ref, tb_ref, s1_ref, s2_ref,
         sew1_ref, seb1_ref, sew2_ref, seb2_ref, fcw_ref, fcb_ref,
         e_ref, hsn_ref, out_ref) = refs
    else:
        (degw_ref, f_ref, bc_ref, br_ref, sseg_ref, cnt_ref, wn_ref,
         tW_ref, tWbT_ref, tb_ref,
         e_ref, hsn_ref, out_ref) = refs
    i = pl.program_id(0)
    iota_bg = lax.broadcasted_iota(jnp.int32, (B, G), 1)
    iota_gb = lax.broadcasted_iota(jnp.int32, (G, B), 0)

    @pl.when(i == 0)
    def _():
        e_ref[...] = jnp.zeros_like(e_ref)

    for g in range(2):
        f = f_ref[g]
        tg = jnp.tanh(sseg_ref[g] / jnp.maximum(cnt_ref[g], 1.0))
        P = (jnp.broadcast_to(bc_ref[g, 0], (B, G)) == iota_bg
             ).astype(jnp.float32)
        Pt = (jnp.broadcast_to(br_ref[g, 0], (G, B)) == iota_gb
              ).astype(jnp.float32)
        tgn = _dot(P, tg)
        coefs_d = jax.nn.sigmoid(
            _dot(f * tgn, jnp.ones((d, d), jnp.float32)) * 10.0)
        e_ref[g] += _dot(Pt, coefs_d * f)
        if dnext:
            dinv_dn = _dot(lax.rsqrt(degw_ref[g] + 1.0), _col0_to(dnext))
            hsn_ref[g] = dinv_dn * _dot(f, wn_ref[...])

    @pl.when(i == NB - 1)
    def _():
        s = _ntn(e_ref[0], e_ref[1], tW_ref[...], tWbT_ref[...],
                 tb_ref[...], d, d // 2)
        if final:
            scores = jnp.concatenate([s, s2_ref[...], s1_ref[...]], axis=1)
            se = jax.nn.sigmoid(
                _dot(jnp.maximum(_dot(scores, sew1_ref[...]) + seb1_ref[...],
                                 0.0), sew2_ref[...]) + seb2_ref[...])
            out_ref[...] = jnp.maximum(
                _dot(se * scores + scores, fcw_ref[...]) + fcb_ref[...], 0.0)
        else:
            out_ref[...] = s


@functools.partial(jax.jit, static_argnames=("d", "dnext", "final"))
def _tc_layer_b(degw, f, bc3, br3, sseg, cnt, Wnext, tWf, tWbT, tb,
                extras, d, dnext, final):
    dn = dnext or 8
    dh = d // 2
    in_specs = [pl.BlockSpec((2, B, 16), lambda i: (0, i, 0)),
                _row2_spec(d),
                pl.BlockSpec((2, 1, B, 1), lambda i: (0, i, 0, 0)),
                pl.BlockSpec((2, 1, 1, B), lambda i: (0, i, 0, 0)),
                _full_spec((2, G, d)), _full_spec((2, G, 1)),
                _full_spec((d, dn)),
                _full_spec((d, d * dh)), _full_spec((2 * d, dh)),
                _full_spec((1, dh))]
    args = [degw, f, bc3, br3, sseg, cnt, Wnext, tWf, tWbT, tb]
    if final:
        in_specs += [_full_spec(x.shape) for x in extras]
        args += list(extras)
    nout = 64 + 48 if final else dh
    out_specs = [_full_spec((2, G, d)), _row2_spec(dn),
                 _full_spec((G, 64 if final else dh))]
    out_shape = [jax.ShapeDtypeStruct((2, G, d), jnp.float32),
                 jax.ShapeDtypeStruct((2, N, dn), jnp.float32),
                 jax.ShapeDtypeStruct((G, 64 if final else dh),
                                      jnp.float32)]
    return pl.pallas_call(
        functools.partial(_layer_b_body, d, dnext, final),
        grid=(NB,),
        in_specs=in_specs,
        out_specs=out_specs,
        out_shape=out_shape,
    )(*args)


def kernel(edge_index_1, features_1, batch_1, i_1, edge_index_2, features_2,
           batch_2, i_2, W1, b1, W2, b2, W3, b3, a1w1, a1b1, a1w2, a1b2,
           a2w1, a2b1, a2w2, a2b2, a3w1, a3b1, a3w2, a3b2, t1W, t1Wb, t1b,
           t2W, t2Wb, t2b, t3W, t3Wb, t3b, fc_w, fc_b, se_w1, se_b1,
           se_w2, se_b2):
    src_flat = jnp.concatenate([edge_index_1[0], edge_index_2[0] + N])
    dst_flat = jnp.concatenate([edge_index_1[1], edge_index_2[1]])
    batch = jnp.stack([batch_1, batch_2])
    bc3 = batch.reshape(2, NB, B, 1)
    br3 = batch.reshape(2, NB, 1, B)
    tWf = (t1W.reshape(F1, F1 * (F1 // 2)),
           t2W.reshape(F2, F2 * (F2 // 2)),
           t3W.reshape(F3, F3 * (F3 // 2)))
    tWbT = (t1Wb.T, t2Wb.T, t3Wb.T)
    tb = (t1b.reshape(1, -1), t2b.reshape(1, -1), t3b.reshape(1, -1))
    aws = ((a1w1, a1b1.reshape(1, -1), a1w2, a1b2.reshape(1, -1)),
           (a2w1, a2b1.reshape(1, -1), a2w2, a2b2.reshape(1, -1)),
           (a3w1, a3b1.reshape(1, -1), a3w2, a3b2.reshape(1, -1)))
    bs = (b1.reshape(1, F1), b2.reshape(1, F2), b3.reshape(1, F3))
    Wn = (W2, W3, None)
    dims = (F1, F2, F3)
    ones_rows = jnp.ones((ECH, 16), jnp.float32)
    z16 = jnp.zeros((RPT, 16), jnp.float32)
    zd = {dd: jnp.zeros((RPT, dd), jnp.float32) for dd in dims}

    degw = _sc_degree(dst_flat, ones_rows, z16)
    hs = _tc_prep(degw, features_1, features_2, W1)

    ss = []
    out = None
    for l in range(3):
        d, dnext = dims[l], (dims[l + 1] if l < 2 else 0)
        acc = _sc_edge_pass(hs.reshape(2 * N, d), src_flat, dst_flat,
                            zd[d], d=d, kdepth=1 if d == F1 else 4)
        aw1, ab1, aw2, ab2 = aws[l]
        f, sseg, cnt = _tc_layer_a(degw, acc, hs, bs[l], br3,
                                   aw1, ab1, aw2, ab2, d=d)
        wn = Wn[l] if Wn[l] is not None else jnp.zeros((d, 8), jnp.float32)
        final = l == 2
        extras = ((ss[0], ss[1], se_w1, se_b1.reshape(1, -1), se_w2,
                   se_b2.reshape(1, -1), fc_w, fc_b.reshape(1, -1))
                  if final else ())
        e, hsn, out_l = _tc_layer_b(degw, f, bc3, br3, sseg, cnt, wn,
                                    tWf[l], tWbT[l], tb[l], extras,
                                    d=d, dnext=dnext, final=final)
        if final:
            out = out_l
        else:
            ss.append(out_l)
            hs = hsn
    return out


# speedup vs baseline: 29.4806x; 1.4086x over previous
import functools

import jax
import jax.numpy as jnp
from jax import lax
from jax.experimental import pallas as pl
from jax.experimental.pallas import tpu as pltpu
from jax.experimental.pallas import tpu_sc as plsc

N = 10000
E = 320000
G = 100
F1, F2, F3 = 128, 64, 32

SC_CORES = 2
SC_TILES = 16
EPT = E // SC_TILES
ECH = 125
TCH = EPT // ECH
NP = 10240
RPT = NP // SC_TILES

NB = 5
B = N // NB

def _dot(a, b):
    return jnp.dot(a, b, preferred_element_type=jnp.float32)


def _col0_to(width):
    r = lax.broadcasted_iota(jnp.int32, (16, width), 0)
    return (r == 0).astype(jnp.float32)


def _sc_mesh():
    return plsc.VectorSubcoreMesh(
        core_axis_name="c", subcore_axis_name="s",
        num_cores=SC_CORES, num_subcores=SC_TILES)


@functools.partial(jax.jit, static_argnames=("d", "kdepth"))
def _sc_edge_pass(hs_table, src_flat, dst_flat, zeros_tile, d, kdepth=1):
    K = kdepth
    NSLOT = 2 * K
    BCH = 16
    NBLK = TCH // BCH
    assert TCH % NSLOT == 0 and BCH % NSLOT == 0 and BCH > 2 * K
    src_r = src_flat.reshape(SC_CORES * SC_TILES, NBLK, BCH, ECH)
    dst_r = dst_flat.reshape(SC_CORES * SC_TILES, NBLK, BCH, ECH)

    @functools.partial(
        pl.kernel,
        out_type=jax.ShapeDtypeStruct((SC_CORES, NP, d), jnp.float32),
        mesh=_sc_mesh(),
        compiler_params=pltpu.CompilerParams(use_tc_tiling_on_sc=False),
        scratch_types=[
            pltpu.VMEM((2, BCH, ECH), jnp.int32),
            pltpu.VMEM((2, BCH, ECH), jnp.int32),
            pltpu.VMEM((NSLOT, ECH, d), jnp.float32),
            pltpu.VMEM_SHARED((NP, d), jnp.float32),
        ] + [pltpu.SemaphoreType.DMA] * (2 * NSLOT + 1),
    )
    def k(hs_hbm, src_hbm, dst_hbm, zero_hbm, out_hbm, src_v, dst_v, rows_v,
          acc_sh, *sems):
        gsem = sems[:NSLOT]
        ssem = sems[NSLOT:2 * NSLOT]
        isem = sems[2 * NSLOT]
        c = lax.axis_index("c")
        s = lax.axis_index("s")
        w = c * SC_TILES + s
        pltpu.sync_copy(zero_hbm, acc_sh.at[pl.ds(s * RPT, RPT)])
        pltpu.sync_copy(src_hbm.at[w, 0], src_v.at[0])
        pltpu.sync_copy(dst_hbm.at[w, 0], dst_v.at[0])
        plsc.subcore_barrier()

        for j in range(K):
            pltpu.async_copy(hs_hbm.at[src_v.at[0].at[j]], rows_v.at[j],
                             gsem[j])

        def body(tt, carry):
            for j in range(NSLOT):
                t = tt * NSLOT + j
                jk = (j + K) % NSLOT
                bb = t // BCH
                r = t % BCH
                p = bb % 2
                t2 = t + K
                p2 = (t2 // BCH) % 2
                r2 = t2 % BCH

                @pl.when((r == K) & (bb + 1 < NBLK))
                def _():
                    pn = (bb + 1) % 2
                    pltpu.async_copy(src_hbm.at[w, bb + 1], src_v.at[pn],
                                     isem)
                    pltpu.async_copy(dst_hbm.at[w, bb + 1], dst_v.at[pn],
                                     isem)

                @pl.when((r == BCH - K) & (bb + 1 < NBLK))
                def _():
                    pltpu.make_async_copy(src_hbm.at[w, 0], src_v.at[0],
                                          isem).wait()
                    pltpu.make_async_copy(dst_hbm.at[w, 0], dst_v.at[0],
                                          isem).wait()

                pltpu.make_async_copy(
                    hs_hbm.at[src_v.at[p].at[r]], rows_v.at[j],
                    gsem[j]).wait()
                pltpu.async_copy(rows_v.at[j], acc_sh.at[dst_v.at[p].at[r]],
                                 ssem[j], add=True)

                @pl.when(t >= K)
                def _():
                    pltpu.make_async_copy(
                        rows_v.at[jk], acc_sh.at[dst_v.at[p].at[r]],
                        ssem[jk]).wait()

                @pl.when(t2 < TCH)
                def _():
                    pltpu.async_copy(hs_hbm.at[src_v.at[p2].at[r2]],
                                     rows_v.at[jk], gsem[jk])
            return carry

        lax.fori_loop(0, TCH // NSLOT, body, jnp.int32(0))
        for j in range(K, NSLOT):
            pltpu.make_async_copy(
                rows_v.at[j], acc_sh.at[dst_v.at[0].at[0]], ssem[j]).wait()
        plsc.subcore_barrier()
        pltpu.sync_copy(acc_sh.at[pl.ds(s * RPT, RPT)],
                        out_hbm.at[c, pl.ds(s * RPT, RPT)])

    return k(hs_table, src_r, dst_r, zeros_tile)


@jax.jit
def _sc_degree(dst_flat, ones_rows, zeros_tile):
    dst_r = dst_flat.reshape(SC_CORES * SC_TILES, TCH, ECH)

    @functools.partial(
        pl.kernel,
        out_type=jax.ShapeDtypeStruct((SC_CORES, NP, 16), jnp.float32),
        mesh=_sc_mesh(),
        compiler_params=pltpu.CompilerParams(use_tc_tiling_on_sc=False),
        scratch_types=[
            pltpu.VMEM((TCH, ECH), jnp.int32),
            pltpu.VMEM((ECH, 16), jnp.float32),
            pltpu.VMEM_SHARED((NP, 16), jnp.float32),
            pltpu.SemaphoreType.DMA,
        ],
    )
    def k(dst_hbm, ones_hbm, zero_hbm, out_hbm, dst_v, ones_v, acc_sh, ssem):
        c = lax.axis_index("c")
        s = lax.axis_index("s")
        w = c * SC_TILES + s
        pltpu.sync_copy(ones_hbm, ones_v)
        pltpu.sync_copy(dst_hbm.at[w], dst_v)
        pltpu.sync_copy(zero_hbm, acc_sh.at[pl.ds(s * RPT, RPT)])
        plsc.subcore_barrier()

        def body(t, carry):
            pltpu.async_copy(ones_v, acc_sh.at[dst_v.at[t]], ssem, add=True)
            return carry

        lax.fori_loop(0, TCH, body, jnp.int32(0))

        def drain(t, carry):
            pltpu.make_async_copy(ones_v, acc_sh.at[dst_v.at[0]],
                                  ssem).wait()
            return carry

        lax.fori_loop(0, TCH, drain, jnp.int32(0))
        plsc.subcore_barrier()
        pltpu.sync_copy(acc_sh.at[pl.ds(s * RPT, RPT)],
                        out_hbm.at[c, pl.ds(s * RPT, RPT)])

    return k(dst_r, ones_rows, zeros_tile)


def _row2_spec(d):
    return pl.BlockSpec((2, B, d), lambda i: (0, i, 0))


def _full_spec(shape):
    nd = len(shape)
    return pl.BlockSpec(shape, lambda i: (0,) * nd)


def _prep_body(degw_ref, x1_ref, x2_ref, w1_ref, hs_ref):
    for g, x_ref in ((0, x1_ref), (1, x2_ref)):
        dinv_d = _dot(lax.rsqrt(degw_ref[g] + 1.0), _col0_to(F1))
        hs_ref[g] = dinv_d * _dot(x_ref[...], w1_ref[...])


@jax.jit
def _tc_prep(degw, x1, x2, W1):
    return pl.pallas_call(
        _prep_body,
        grid=(NB,),
        in_specs=[pl.BlockSpec((2, B, 16), lambda i: (0, i, 0)),
                  pl.BlockSpec((B, F1), lambda i: (i, 0)),
                  pl.BlockSpec((B, F1), lambda i: (i, 0)),
                  _full_spec((F1, F1))],
        out_specs=_row2_spec(F1),
        out_shape=jax.ShapeDtypeStruct((2, N, F1), jnp.float32),
    )(degw, x1, x2, W1)


def _layer_a_body(d, degw_ref, acc_ref, hs_ref, b_ref, br_ref,
                  aw1_ref, ab1_ref, aw2_ref, ab2_ref,
                  f_ref, sseg_ref, cnt_ref):
    i = pl.program_id(0)
    iota_gb = lax.broadcasted_iota(jnp.int32, (G, B), 0)

    @pl.when(i == 0)
    def _():
        sseg_ref[...] = jnp.zeros_like(sseg_ref)
        cnt_ref[...] = jnp.zeros_like(cnt_ref)

    for g in range(2):
        dinv_d = _dot(lax.rsqrt(degw_ref[g] + 1.0), _col0_to(d))
        f = jnp.maximum(dinv_d * (acc_ref[g] + hs_ref[g]) + b_ref[...], 0.0)
        f_ref[g] = f
        u = jnp.maximum(_dot(f, aw1_ref[...]) + ab1_ref[...], 0.0)
        att = jnp.tanh(_dot(u, aw2_ref[...]) + ab2_ref[...])
        Pt = (jnp.broadcast_to(br_ref[g, 0], (G, B)) == iota_gb
              ).astype(jnp.float32)
        sseg_ref[g] += _dot(Pt, f * att)
        cnt_ref[g] += _dot(Pt, jnp.ones((B, 1), jnp.float32))


@functools.partial(jax.jit, static_argnames=("d",))
def _tc_layer_a(degw, acc, hs, b, br3, aw1, ab1, aw2, ab2, d):
    r = aw1.shape[1]
    return pl.pallas_call(
        functools.partial(_layer_a_body, d),
        grid=(NB,),
        in_specs=[pl.BlockSpec((2, B, 16), lambda i: (0, i, 0)),
                  _row2_spec(d), _row2_spec(d),
                  _full_spec((1, d)),
                  pl.BlockSpec((2, 1, 1, B), lambda i: (0, i, 0, 0)),
                  _full_spec((d, r)), _full_spec((1, r)),
                  _full_spec((r, d)), _full_spec((1, d))],
        out_specs=[_row2_spec(d), _full_spec((2, G, d)),
                   _full_spec((2, G, 1))],
        out_shape=[jax.ShapeDtypeStruct((2, N, d), jnp.float32),
                   jax.ShapeDtypeStruct((2, G, d), jnp.float32),
                   jax.ShapeDtypeStruct((2, G, 1), jnp.float32)],
    )(degw, acc, hs, b, br3, aw1, ab1, aw2, ab2)


def _ntn(e1, e2, tW, tWbT, tb, d, dh):
    T3 = _dot(e1, tW)
    scoring = jnp.sum(T3.reshape(G, d, dh) * e2[:, :, None], axis=1)
    block = _dot(jnp.concatenate([e1, e2], axis=1), tWbT)
    return jnp.maximum(scoring + block + tb, 0.0)


def _layer_b_body(d, dnext, final, *refs):
    if final:
        (degw_ref, f_ref, bc_ref, br_ref, sseg_ref, cnt_ref, wn_ref,
         tW_ref, tWbT_ref, tb_ref, s1_ref, s2_ref,
         sew1_ref, seb1_ref, sew2_ref, seb2_ref, fcw_ref, fcb_ref,
         e_ref, hsn_ref, out_ref) = refs
    else:
        (degw_ref, f_ref, bc_ref, br_ref, sseg_ref, cnt_ref, wn_ref,
         tW_ref, tWbT_ref, tb_ref,
         e_ref, hsn_ref, out_ref) = refs
    i = pl.program_id(0)
    iota_bg = lax.broadcasted_iota(jnp.int32, (B, G), 1)
    iota_gb = lax.broadcasted_iota(jnp.int32, (G, B), 0)

    @pl.when(i == 0)
    def _():
        e_ref[...] = jnp.zeros_like(e_ref)

    for g in range(2):
        f = f_ref[g]
        tg = jnp.tanh(sseg_ref[g] / jnp.maximum(cnt_ref[g], 1.0))
        P = (jnp.broadcast_to(bc_ref[g, 0], (B, G)) == iota_bg
             ).astype(jnp.float32)
        Pt = (jnp.broadcast_to(br_ref[g, 0], (G, B)) == iota_gb
              ).astype(jnp.float32)
        tgn = _dot(P, tg)
        coefs_d = jax.nn.sigmoid(
            _dot(f * tgn, jnp.ones((d, d), jnp.float32)) * 10.0)
        e_ref[g] += _dot(Pt, coefs_d * f)
        if dnext:
            dinv_dn = _dot(lax.rsqrt(degw_ref[g] + 1.0), _col0_to(dnext))
            hsn_ref[g] = dinv_dn * _dot(f, wn_ref[...])

    @pl.when(i == NB - 1)
    def _():
        s = _ntn(e_ref[0], e_ref[1], tW_ref[...], tWbT_ref[...],
                 tb_ref[...], d, d // 2)
        if final:
            scores = jnp.concatenate([s, s2_ref[...], s1_ref[...]], axis=1)
            se = jax.nn.sigmoid(
                _dot(jnp.maximum(_dot(scores, sew1_ref[...]) + seb1_ref[...],
                                 0.0), sew2_ref[...]) + seb2_ref[...])
            out_ref[...] = jnp.maximum(
                _dot(se * scores + scores, fcw_ref[...]) + fcb_ref[...], 0.0)
        else:
            out_ref[...] = s


@functools.partial(jax.jit, static_argnames=("d", "dnext", "final"))
def _tc_layer_b(degw, f, bc3, br3, sseg, cnt, Wnext, tWf, tWbT, tb,
                extras, d, dnext, final):
    dn = dnext or 8
    dh = d // 2
    in_specs = [pl.BlockSpec((2, B, 16), lambda i: (0, i, 0)),
                _row2_spec(d),
                pl.BlockSpec((2, 1, B, 1), lambda i: (0, i, 0, 0)),
                pl.BlockSpec((2, 1, 1, B), lambda i: (0, i, 0, 0)),
                _full_spec((2, G, d)), _full_spec((2, G, 1)),
                _full_spec((d, dn)),
                _full_spec((d, d * dh)), _full_spec((2 * d, dh)),
                _full_spec((1, dh))]
    args = [degw, f, bc3, br3, sseg, cnt, Wnext, tWf, tWbT, tb]
    if final:
        in_specs += [_full_spec(x.shape) for x in extras]
        args += list(extras)
    nout = 64 + 48 if final else dh
    out_specs = [_full_spec((2, G, d)), _row2_spec(dn),
                 _full_spec((G, 64 if final else dh))]
    out_shape = [jax.ShapeDtypeStruct((2, G, d), jnp.float32),
                 jax.ShapeDtypeStruct((2, N, dn), jnp.float32),
                 jax.ShapeDtypeStruct((G, 64 if final else dh),
                                      jnp.float32)]
    return pl.pallas_call(
        functools.partial(_layer_b_body, d, dnext, final),
        grid=(NB,),
        in_specs=in_specs,
        out_specs=out_specs,
        out_shape=out_shape,
    )(*args)


def kernel(edge_index_1, features_1, batch_1, i_1, edge_index_2, features_2,
           batch_2, i_2, W1, b1, W2, b2, W3, b3, a1w1, a1b1, a1w2, a1b2,
           a2w1, a2b1, a2w2, a2b2, a3w1, a3b1, a3w2, a3b2, t1W, t1Wb, t1b,
           t2W, t2Wb, t2b, t3W, t3Wb, t3b, fc_w, fc_b, se_w1, se_b1,
           se_w2, se_b2):
    src_flat = jnp.concatenate([edge_index_1[0], edge_index_2[0] + N])
    dst_flat = jnp.concatenate([edge_index_1[1], edge_index_2[1]])
    batch = jnp.stack([batch_1, batch_2])
    bc3 = batch.reshape(2, NB, B, 1)
    br3 = batch.reshape(2, NB, 1, B)
    tWf = (t1W.reshape(F1, F1 * (F1 // 2)),
           t2W.reshape(F2, F2 * (F2 // 2)),
           t3W.reshape(F3, F3 * (F3 // 2)))
    tWbT = (t1Wb.T, t2Wb.T, t3Wb.T)
    tb = (t1b.reshape(1, -1), t2b.reshape(1, -1), t3b.reshape(1, -1))
    aws = ((a1w1, a1b1.reshape(1, -1), a1w2, a1b2.reshape(1, -1)),
           (a2w1, a2b1.reshape(1, -1), a2w2, a2b2.reshape(1, -1)),
           (a3w1, a3b1.reshape(1, -1), a3w2, a3b2.reshape(1, -1)))
    bs = (b1.reshape(1, F1), b2.reshape(1, F2), b3.reshape(1, F3))
    Wn = (W2, W3, None)
    dims = (F1, F2, F3)
    ones_rows = jnp.ones((ECH, 16), jnp.float32)
    z16 = jnp.zeros((RPT, 16), jnp.float32)
    zd = {dd: jnp.zeros((RPT, dd), jnp.float32) for dd in dims}

    degw = _sc_degree(dst_flat, ones_rows, z16)
    hs = _tc_prep(degw, features_1, features_2, W1)

    ss = []
    out = None
    for l in range(3):
        d, dnext = dims[l], (dims[l + 1] if l < 2 else 0)
        acc = _sc_edge_pass(hs.reshape(2 * N, d), src_flat, dst_flat,
                            zd[d], d=d, kdepth=1 if d == F1 else 4)
        aw1, ab1, aw2, ab2 = aws[l]
        f, sseg, cnt = _tc_layer_a(degw, acc, hs, bs[l], br3,
                                   aw1, ab1, aw2, ab2, d=d)
        wn = Wn[l] if Wn[l] is not None else jnp.zeros((d, 8), jnp.float32)
        final = l == 2
        extras = ((ss[0], ss[1], se_w1, se_b1.reshape(1, -1), se_w2,
                   se_b2.reshape(1, -1), fc_w, fc_b.reshape(1, -1))
                  if final else ())
        e, hsn, out_l = _tc_layer_b(degw, f, bc3, br3, sseg, cnt, wn,
                                    tWf[l], tWbT[l], tb[l], extras,
                                    d=d, dnext=dnext, final=final)
        if final:
            out = out_l
        else:
            ss.append(out_l)
            hs = hsn
    return out


# speedup vs baseline: 34.6646x; 1.1758x over previous
import functools

import jax
import jax.numpy as jnp
from jax import lax
from jax.experimental import pallas as pl
from jax.experimental.pallas import tpu as pltpu
from jax.experimental.pallas import tpu_sc as plsc

N = 10000
E = 320000
G = 100
F1, F2, F3 = 128, 64, 32

SC_CORES = 2
SC_TILES = 16
EPT = E // SC_TILES
ECH = 125
TCH = EPT // ECH
NP = 10240
RPT = NP // SC_TILES

NB = 5
B = N // NB

def _dot(a, b):
    return jnp.dot(a, b, preferred_element_type=jnp.float32)


def _col0_to(width):
    r = lax.broadcasted_iota(jnp.int32, (16, width), 0)
    return (r == 0).astype(jnp.float32)


def _sc_mesh():
    return plsc.VectorSubcoreMesh(
        core_axis_name="c", subcore_axis_name="s",
        num_cores=SC_CORES, num_subcores=SC_TILES)


@functools.partial(jax.jit, static_argnames=("d", "kdepth"))
def _sc_edge_pass(hs_table, src_flat, dst_flat, zeros_tile, d, kdepth=1):
    K = kdepth
    NSLOT = 2 * K
    BCH = 16
    NBLK = TCH // BCH
    assert TCH % NSLOT == 0 and BCH % NSLOT == 0 and BCH > 2 * K
    src_r = src_flat.reshape(SC_CORES * SC_TILES, NBLK, BCH, ECH)
    dst_r = dst_flat.reshape(SC_CORES * SC_TILES, NBLK, BCH, ECH)

    @functools.partial(
        pl.kernel,
        out_type=jax.ShapeDtypeStruct((SC_CORES, NP, d), jnp.bfloat16),
        mesh=_sc_mesh(),
        compiler_params=pltpu.CompilerParams(use_tc_tiling_on_sc=False),
        scratch_types=[
            pltpu.VMEM((2, BCH, ECH), jnp.int32),
            pltpu.VMEM((2, BCH, ECH), jnp.int32),
            pltpu.VMEM((NSLOT, ECH, d), jnp.bfloat16),
            pltpu.VMEM_SHARED((NP, d), jnp.bfloat16),
        ] + [pltpu.SemaphoreType.DMA] * (2 * NSLOT + 1),
    )
    def k(hs_hbm, src_hbm, dst_hbm, zero_hbm, out_hbm, src_v, dst_v, rows_v,
          acc_sh, *sems):
        gsem = sems[:NSLOT]
        ssem = sems[NSLOT:2 * NSLOT]
        isem = sems[2 * NSLOT]
        c = lax.axis_index("c")
        s = lax.axis_index("s")
        w = c * SC_TILES + s
        pltpu.sync_copy(zero_hbm, acc_sh.at[pl.ds(s * RPT, RPT)])
        pltpu.sync_copy(src_hbm.at[w, 0], src_v.at[0])
        pltpu.sync_copy(dst_hbm.at[w, 0], dst_v.at[0])
        plsc.subcore_barrier()

        for j in range(K):
            pltpu.async_copy(hs_hbm.at[src_v.at[0].at[j]], rows_v.at[j],
                             gsem[j])

        def body(tt, carry):
            for j in range(NSLOT):
                t = tt * NSLOT + j
                jk = (j + K) % NSLOT
                bb = t // BCH
                r = t % BCH
                p = bb % 2
                t2 = t + K
                p2 = (t2 // BCH) % 2
                r2 = t2 % BCH

                @pl.when((r == K) & (bb + 1 < NBLK))
                def _():
                    pn = (bb + 1) % 2
                    pltpu.async_copy(src_hbm.at[w, bb + 1], src_v.at[pn],
                                     isem)
                    pltpu.async_copy(dst_hbm.at[w, bb + 1], dst_v.at[pn],
                                     isem)

                @pl.when((r == BCH - K) & (bb + 1 < NBLK))
                def _():
                    pltpu.make_async_copy(src_hbm.at[w, 0], src_v.at[0],
                                          isem).wait()
                    pltpu.make_async_copy(dst_hbm.at[w, 0], dst_v.at[0],
                                          isem).wait()

                pltpu.make_async_copy(
                    hs_hbm.at[src_v.at[p].at[r]], rows_v.at[j],
                    gsem[j]).wait()
                pltpu.async_copy(rows_v.at[j], acc_sh.at[dst_v.at[p].at[r]],
                                 ssem[j], add=True)

                @pl.when(t >= K)
                def _():
                    pltpu.make_async_copy(
                        rows_v.at[jk], acc_sh.at[dst_v.at[p].at[r]],
                        ssem[jk]).wait()

                @pl.when(t2 < TCH)
                def _():
                    pltpu.async_copy(hs_hbm.at[src_v.at[p2].at[r2]],
                                     rows_v.at[jk], gsem[jk])
            return carry

        lax.fori_loop(0, TCH // NSLOT, body, jnp.int32(0))
        for j in range(K, NSLOT):
            pltpu.make_async_copy(
                rows_v.at[j], acc_sh.at[dst_v.at[0].at[0]], ssem[j]).wait()
        plsc.subcore_barrier()
        pltpu.sync_copy(acc_sh.at[pl.ds(s * RPT, RPT)],
                        out_hbm.at[c, pl.ds(s * RPT, RPT)])

    return k(hs_table, src_r, dst_r, zeros_tile)


@jax.jit
def _sc_degree(dst_flat, ones_rows, zeros_tile):
    dst_r = dst_flat.reshape(SC_CORES * SC_TILES, TCH, ECH)

    @functools.partial(
        pl.kernel,
        out_type=jax.ShapeDtypeStruct((SC_CORES, NP, 16), jnp.float32),
        mesh=_sc_mesh(),
        compiler_params=pltpu.CompilerParams(use_tc_tiling_on_sc=False),
        scratch_types=[
            pltpu.VMEM((TCH, ECH), jnp.int32),
            pltpu.VMEM((ECH, 16), jnp.float32),
            pltpu.VMEM_SHARED((NP, 16), jnp.float32),
            pltpu.SemaphoreType.DMA,
        ],
    )
    def k(dst_hbm, ones_hbm, zero_hbm, out_hbm, dst_v, ones_v, acc_sh, ssem):
        c = lax.axis_index("c")
        s = lax.axis_index("s")
        w = c * SC_TILES + s
        pltpu.sync_copy(ones_hbm, ones_v)
        pltpu.sync_copy(dst_hbm.at[w], dst_v)
        pltpu.sync_copy(zero_hbm, acc_sh.at[pl.ds(s * RPT, RPT)])
        plsc.subcore_barrier()

        def body(t, carry):
            pltpu.async_copy(ones_v, acc_sh.at[dst_v.at[t]], ssem, add=True)
            return carry

        lax.fori_loop(0, TCH, body, jnp.int32(0))

        def drain(t, carry):
            pltpu.make_async_copy(ones_v, acc_sh.at[dst_v.at[0]],
                                  ssem).wait()
            return carry

        lax.fori_loop(0, TCH, drain, jnp.int32(0))
        plsc.subcore_barrier()
        pltpu.sync_copy(acc_sh.at[pl.ds(s * RPT, RPT)],
                        out_hbm.at[c, pl.ds(s * RPT, RPT)])

    return k(dst_r, ones_rows, zeros_tile)


def _row2_spec(d):
    return pl.BlockSpec((2, B, d), lambda i: (0, i, 0))


def _full_spec(shape):
    nd = len(shape)
    return pl.BlockSpec(shape, lambda i: (0,) * nd)


def _prep_body(degw_ref, x1_ref, x2_ref, w1_ref, hs_ref):
    for g, x_ref in ((0, x1_ref), (1, x2_ref)):
        dinv_d = _dot(lax.rsqrt(degw_ref[g] + 1.0), _col0_to(F1))
        hs_ref[g] = (dinv_d * _dot(x_ref[...], w1_ref[...])
                     ).astype(jnp.bfloat16)


@jax.jit
def _tc_prep(degw, x1, x2, W1):
    return pl.pallas_call(
        _prep_body,
        grid=(NB,),
        in_specs=[pl.BlockSpec((2, B, 16), lambda i: (0, i, 0)),
                  pl.BlockSpec((B, F1), lambda i: (i, 0)),
                  pl.BlockSpec((B, F1), lambda i: (i, 0)),
                  _full_spec((F1, F1))],
        out_specs=_row2_spec(F1),
        out_shape=jax.ShapeDtypeStruct((2, N, F1), jnp.bfloat16),
    )(degw, x1, x2, W1)


def _layer_a_body(d, degw_ref, acc_ref, hs_ref, b_ref, br_ref,
                  aw1_ref, ab1_ref, aw2_ref, ab2_ref,
                  f_ref, sseg_ref, cnt_ref):
    i = pl.program_id(0)
    iota_gb = lax.broadcasted_iota(jnp.int32, (G, B), 0)

    @pl.when(i == 0)
    def _():
        sseg_ref[...] = jnp.zeros_like(sseg_ref)
        cnt_ref[...] = jnp.zeros_like(cnt_ref)

    for g in range(2):
        dinv_d = _dot(lax.rsqrt(degw_ref[g] + 1.0), _col0_to(d))
        f = jnp.maximum(
            dinv_d * (acc_ref[g].astype(jnp.float32) +
                      hs_ref[g].astype(jnp.float32)) + b_ref[...], 0.0)
        f_ref[g] = f
        u = jnp.maximum(_dot(f, aw1_ref[...]) + ab1_ref[...], 0.0)
        att = jnp.tanh(_dot(u, aw2_ref[...]) + ab2_ref[...])
        Pt = (jnp.broadcast_to(br_ref[g, 0], (G, B)) == iota_gb
              ).astype(jnp.float32)
        sseg_ref[g] += _dot(Pt, f * att)
        cnt_ref[g] += _dot(Pt, jnp.ones((B, 1), jnp.float32))


@functools.partial(jax.jit, static_argnames=("d",))
def _tc_layer_a(degw, acc, hs, b, br3, aw1, ab1, aw2, ab2, d):
    r = aw1.shape[1]
    return pl.pallas_call(
        functools.partial(_layer_a_body, d),
        grid=(NB,),
        in_specs=[pl.BlockSpec((2, B, 16), lambda i: (0, i, 0)),
                  _row2_spec(d), _row2_spec(d),
                  _full_spec((1, d)),
                  pl.BlockSpec((2, 1, 1, B), lambda i: (0, i, 0, 0)),
                  _full_spec((d, r)), _full_spec((1, r)),
                  _full_spec((r, d)), _full_spec((1, d))],
        out_specs=[_row2_spec(d), _full_spec((2, G, d)),
                   _full_spec((2, G, 1))],
        out_shape=[jax.ShapeDtypeStruct((2, N, d), jnp.float32),
                   jax.ShapeDtypeStruct((2, G, d), jnp.float32),
                   jax.ShapeDtypeStruct((2, G, 1), jnp.float32)],
    )(degw, acc, hs, b, br3, aw1, ab1, aw2, ab2)


def _ntn(e1, e2, tW, tWbT, tb, d, dh):
    T3 = _dot(e1, tW)
    scoring = jnp.sum(T3.reshape(G, d, dh) * e2[:, :, None], axis=1)
    block = _dot(jnp.concatenate([e1, e2], axis=1), tWbT)
    return jnp.maximum(scoring + block + tb, 0.0)


def _layer_b_body(d, dnext, final, *refs):
    if final:
        (degw_ref, f_ref, bc_ref, br_ref, sseg_ref, cnt_ref, wn_ref,
         tW_ref, tWbT_ref, tb_ref, s1_ref, s2_ref,
         sew1_ref, seb1_ref, sew2_ref, seb2_ref, fcw_ref, fcb_ref,
         e_ref, hsn_ref, out_ref) = refs
    else:
        (degw_ref, f_ref, bc_ref, br_ref, sseg_ref, cnt_ref, wn_ref,
         tW_ref, tWbT_ref, tb_ref,
         e_ref, hsn_ref, out_ref) = refs
    i = pl.program_id(0)
    iota_bg = lax.broadcasted_iota(jnp.int32, (B, G), 1)
    iota_gb = lax.broadcasted_iota(jnp.int32, (G, B), 0)

    @pl.when(i == 0)
    def _():
        e_ref[...] = jnp.zeros_like(e_ref)

    for g in range(2):
        f = f_ref[g]
        tg = jnp.tanh(sseg_ref[g] / jnp.maximum(cnt_ref[g], 1.0))
        P = (jnp.broadcast_to(bc_ref[g, 0], (B, G)) == iota_bg
             ).astype(jnp.float32)
        Pt = (jnp.broadcast_to(br_ref[g, 0], (G, B)) == iota_gb
              ).astype(jnp.float32)
        tgn = _dot(P, tg)
        coefs_d = jax.nn.sigmoid(
            _dot(f * tgn, jnp.ones((d, d), jnp.float32)) * 10.0)
        e_ref[g] += _dot(Pt, coefs_d * f)
        if dnext:
            dinv_dn = _dot(lax.rsqrt(degw_ref[g] + 1.0), _col0_to(dnext))
            hsn_ref[g] = (dinv_dn * _dot(f, wn_ref[...])
                          ).astype(jnp.bfloat16)

    @pl.when(i == NB - 1)
    def _():
        s = _ntn(e_ref[0], e_ref[1], tW_ref[...], tWbT_ref[...],
                 tb_ref[...], d, d // 2)
        if final:
            scores = jnp.concatenate([s, s2_ref[...], s1_ref[...]], axis=1)
            se = jax.nn.sigmoid(
                _dot(jnp.maximum(_dot(scores, sew1_ref[...]) + seb1_ref[...],
                                 0.0), sew2_ref[...]) + seb2_ref[...])
            out_ref[...] = jnp.maximum(
                _dot(se * scores + scores, fcw_ref[...]) + fcb_ref[...], 0.0)
        else:
            out_ref[...] = s


@functools.partial(jax.jit, static_argnames=("d", "dnext", "final"))
def _tc_layer_b(degw, f, bc3, br3, sseg, cnt, Wnext, tWf, tWbT, tb,
                extras, d, dnext, final):
    dn = dnext or 8
    dh = d // 2
    in_specs = [pl.BlockSpec((2, B, 16), lambda i: (0, i, 0)),
                _row2_spec(d),
                pl.BlockSpec((2, 1, B, 1), lambda i: (0, i, 0, 0)),
                pl.BlockSpec((2, 1, 1, B), lambda i: (0, i, 0, 0)),
                _full_spec((2, G, d)), _full_spec((2, G, 1)),
                _full_spec((d, dn)),
                _full_spec((d, d * dh)), _full_spec((2 * d, dh)),
                _full_spec((1, dh))]
    args = [degw, f, bc3, br3, sseg, cnt, Wnext, tWf, tWbT, tb]
    if final:
        in_specs += [_full_spec(x.shape) for x in extras]
        args += list(extras)
    nout = 64 + 48 if final else dh
    out_specs = [_full_spec((2, G, d)), _row2_spec(dn),
                 _full_spec((G, 64 if final else dh))]
    out_shape = [jax.ShapeDtypeStruct((2, G, d), jnp.float32),
                 jax.ShapeDtypeStruct((2, N, dn), jnp.bfloat16),
                 jax.ShapeDtypeStruct((G, 64 if final else dh),
                                      jnp.float32)]
    return pl.pallas_call(
        functools.partial(_layer_b_body, d, dnext, final),
        grid=(NB,),
        in_specs=in_specs,
        out_specs=out_specs,
        out_shape=out_shape,
    )(*args)


def kernel(edge_index_1, features_1, batch_1, i_1, edge_index_2, features_2,
           batch_2, i_2, W1, b1, W2, b2, W3, b3, a1w1, a1b1, a1w2, a1b2,
           a2w1, a2b1, a2w2, a2b2, a3w1, a3b1, a3w2, a3b2, t1W, t1Wb, t1b,
           t2W, t2Wb, t2b, t3W, t3Wb, t3b, fc_w, fc_b, se_w1, se_b1,
           se_w2, se_b2):
    src_flat = jnp.concatenate([edge_index_1[0], edge_index_2[0] + N])
    dst_flat = jnp.concatenate([edge_index_1[1], edge_index_2[1]])
    batch = jnp.stack([batch_1, batch_2])
    bc3 = batch.reshape(2, NB, B, 1)
    br3 = batch.reshape(2, NB, 1, B)
    tWf = (t1W.reshape(F1, F1 * (F1 // 2)),
           t2W.reshape(F2, F2 * (F2 // 2)),
           t3W.reshape(F3, F3 * (F3 // 2)))
    tWbT = (t1Wb.T, t2Wb.T, t3Wb.T)
    tb = (t1b.reshape(1, -1), t2b.reshape(1, -1), t3b.reshape(1, -1))
    aws = ((a1w1, a1b1.reshape(1, -1), a1w2, a1b2.reshape(1, -1)),
           (a2w1, a2b1.reshape(1, -1), a2w2, a2b2.reshape(1, -1)),
           (a3w1, a3b1.reshape(1, -1), a3w2, a3b2.reshape(1, -1)))
    bs = (b1.reshape(1, F1), b2.reshape(1, F2), b3.reshape(1, F3))
    Wn = (W2, W3, None)
    dims = (F1, F2, F3)
    ones_rows = jnp.ones((ECH, 16), jnp.float32)
    z16 = jnp.zeros((RPT, 16), jnp.float32)
    zd = {dd: jnp.zeros((RPT, dd), jnp.bfloat16) for dd in dims}

    degw = _sc_degree(dst_flat, ones_rows, z16)
    hs = _tc_prep(degw, features_1, features_2, W1)

    ss = []
    out = None
    for l in range(3):
        d, dnext = dims[l], (dims[l + 1] if l < 2 else 0)
        acc = _sc_edge_pass(hs.reshape(2 * N, d), src_flat, dst_flat,
                            zd[d], d=d, kdepth=1 if d == F1 else 4)
        aw1, ab1, aw2, ab2 = aws[l]
        f, sseg, cnt = _tc_layer_a(degw, acc, hs, bs[l], br3,
                                   aw1, ab1, aw2, ab2, d=d)
        wn = Wn[l] if Wn[l] is not None else jnp.zeros((d, 8), jnp.float32)
        final = l == 2
        extras = ((ss[0], ss[1], se_w1, se_b1.reshape(1, -1), se_w2,
                   se_b2.reshape(1, -1), fc_w, fc_b.reshape(1, -1))
                  if final else ())
        e, hsn, out_l = _tc_layer_b(degw, f, bc3, br3, sseg, cnt, wn,
                                    tWf[l], tWbT[l], tb[l], extras,
                                    d=d, dnext=dnext, final=final)
        if final:
            out = out_l
        else:
            ss.append(out_l)
            hs = hsn
    return out


# speedup vs baseline: 39.1023x; 1.1280x over previous
import functools

import jax
import jax.numpy as jnp
from jax import lax
from jax.experimental import pallas as pl
from jax.experimental.pallas import tpu as pltpu
from jax.experimental.pallas import tpu_sc as plsc

N = 10000
E = 320000
G = 100
F1, F2, F3 = 128, 64, 32

SC_CORES = 2
SC_TILES = 16
EPT = E // SC_TILES
ECH = 125
TCH = EPT // ECH
NP = 10240
RPT = NP // SC_TILES

NB = 5
B = N // NB

def _dot(a, b):
    return jnp.dot(a, b, preferred_element_type=jnp.float32)


def _col0_to(width):
    r = lax.broadcasted_iota(jnp.int32, (16, width), 0)
    return (r == 0).astype(jnp.float32)


def _sc_mesh():
    return plsc.VectorSubcoreMesh(
        core_axis_name="c", subcore_axis_name="s",
        num_cores=SC_CORES, num_subcores=SC_TILES)


@functools.partial(jax.jit, static_argnames=("d", "kdepth"))
def _sc_edge_pass(hs_table, src_flat, dst_flat, zeros_tile, d, kdepth=1):
    K = kdepth
    NSLOT = 2 * K
    BCH = 16
    NBLK = TCH // BCH
    assert TCH % NSLOT == 0 and BCH % NSLOT == 0 and BCH > 2 * K
    src_r = src_flat.reshape(SC_CORES * SC_TILES, NBLK, BCH, ECH)
    dst_r = dst_flat.reshape(SC_CORES * SC_TILES, NBLK, BCH, ECH)

    @functools.partial(
        pl.kernel,
        out_type=jax.ShapeDtypeStruct((SC_CORES, NP, d), jnp.bfloat16),
        mesh=_sc_mesh(),
        compiler_params=pltpu.CompilerParams(use_tc_tiling_on_sc=False),
        scratch_types=[
            pltpu.VMEM((2, BCH, ECH), jnp.int32),
            pltpu.VMEM((2, BCH, ECH), jnp.int32),
            pltpu.VMEM((NSLOT, ECH, d), jnp.bfloat16),
            pltpu.VMEM_SHARED((NP, d), jnp.bfloat16),
        ] + [pltpu.SemaphoreType.DMA] * (2 * NSLOT + 1),
    )
    def k(hs_hbm, src_hbm, dst_hbm, zero_hbm, out_hbm, src_v, dst_v, rows_v,
          acc_sh, *sems):
        gsem = sems[:NSLOT]
        ssem = sems[NSLOT:2 * NSLOT]
        isem = sems[2 * NSLOT]
        c = lax.axis_index("c")
        s = lax.axis_index("s")
        w = c * SC_TILES + s
        pltpu.sync_copy(zero_hbm, acc_sh.at[pl.ds(s * RPT, RPT)])
        pltpu.sync_copy(src_hbm.at[w, 0], src_v.at[0])
        pltpu.sync_copy(dst_hbm.at[w, 0], dst_v.at[0])
        plsc.subcore_barrier()

        for j in range(K):
            pltpu.async_copy(hs_hbm.at[src_v.at[0].at[j]], rows_v.at[j],
                             gsem[j])

        def body(tt, carry):
            for j in range(NSLOT):
                t = tt * NSLOT + j
                jk = (j + K) % NSLOT
                bb = t // BCH
                r = t % BCH
                p = bb % 2
                t2 = t + K
                p2 = (t2 // BCH) % 2
                r2 = t2 % BCH

                @pl.when((r == K) & (bb + 1 < NBLK))
                def _():
                    pn = (bb + 1) % 2
                    pltpu.async_copy(src_hbm.at[w, bb + 1], src_v.at[pn],
                                     isem)
                    pltpu.async_copy(dst_hbm.at[w, bb + 1], dst_v.at[pn],
                                     isem)

                @pl.when((r == BCH - K) & (bb + 1 < NBLK))
                def _():
                    pltpu.make_async_copy(src_hbm.at[w, 0], src_v.at[0],
                                          isem).wait()
                    pltpu.make_async_copy(dst_hbm.at[w, 0], dst_v.at[0],
                                          isem).wait()

                pltpu.make_async_copy(
                    hs_hbm.at[src_v.at[p].at[r]], rows_v.at[j],
                    gsem[j]).wait()
                pltpu.async_copy(rows_v.at[j], acc_sh.at[dst_v.at[p].at[r]],
                                 ssem[j], add=True)

                @pl.when(t >= K)
                def _():
                    pltpu.make_async_copy(
                        rows_v.at[jk], acc_sh.at[dst_v.at[p].at[r]],
                        ssem[jk]).wait()

                @pl.when(t2 < TCH)
                def _():
                    pltpu.async_copy(hs_hbm.at[src_v.at[p2].at[r2]],
                                     rows_v.at[jk], gsem[jk])
            return carry

        lax.fori_loop(0, TCH // NSLOT, body, jnp.int32(0))
        for j in range(K, NSLOT):
            pltpu.make_async_copy(
                rows_v.at[j], acc_sh.at[dst_v.at[0].at[0]], ssem[j]).wait()
        plsc.subcore_barrier()
        pltpu.sync_copy(acc_sh.at[pl.ds(s * RPT, RPT)],
                        out_hbm.at[c, pl.ds(s * RPT, RPT)])

    return k(hs_table, src_r, dst_r, zeros_tile)


@jax.jit
def _sc_degree(dst_flat, ones_rows, zeros_tile):
    dst_r = dst_flat.reshape(SC_CORES * SC_TILES, TCH, ECH)

    @functools.partial(
        pl.kernel,
        out_type=jax.ShapeDtypeStruct((SC_CORES, NP, 16), jnp.float32),
        mesh=_sc_mesh(),
        compiler_params=pltpu.CompilerParams(use_tc_tiling_on_sc=False),
        scratch_types=[
            pltpu.VMEM((TCH, ECH), jnp.int32),
            pltpu.VMEM((ECH, 16), jnp.float32),
            pltpu.VMEM_SHARED((NP, 16), jnp.float32),
            pltpu.SemaphoreType.DMA,
        ],
    )
    def k(dst_hbm, ones_hbm, zero_hbm, out_hbm, dst_v, ones_v, acc_sh, ssem):
        c = lax.axis_index("c")
        s = lax.axis_index("s")
        w = c * SC_TILES + s
        pltpu.sync_copy(ones_hbm, ones_v)
        pltpu.sync_copy(dst_hbm.at[w], dst_v)
        pltpu.sync_copy(zero_hbm, acc_sh.at[pl.ds(s * RPT, RPT)])
        plsc.subcore_barrier()

        def body(t, carry):
            pltpu.async_copy(ones_v, acc_sh.at[dst_v.at[t]], ssem, add=True)
            return carry

        lax.fori_loop(0, TCH, body, jnp.int32(0))

        def drain(t, carry):
            pltpu.make_async_copy(ones_v, acc_sh.at[dst_v.at[0]],
                                  ssem).wait()
            return carry

        lax.fori_loop(0, TCH, drain, jnp.int32(0))
        plsc.subcore_barrier()
        pltpu.sync_copy(acc_sh.at[pl.ds(s * RPT, RPT)],
                        out_hbm.at[c, pl.ds(s * RPT, RPT)])

    return k(dst_r, ones_rows, zeros_tile)


def _row2_spec(d):
    return pl.BlockSpec((2, B, d), lambda i: (0, i, 0))


def _full_spec(shape):
    nd = len(shape)
    return pl.BlockSpec(shape, lambda i: (0,) * nd)


def _prep_body(degw_ref, x1_ref, x2_ref, w1_ref, hs_ref):
    for g, x_ref in ((0, x1_ref), (1, x2_ref)):
        dinv_d = _dot(lax.rsqrt(degw_ref[g] + 1.0), _col0_to(F1))
        hs_ref[g] = (dinv_d * _dot(x_ref[...], w1_ref[...])
                     ).astype(jnp.bfloat16)


@jax.jit
def _tc_prep(degw, x1, x2, W1):
    return pl.pallas_call(
        _prep_body,
        grid=(NB,),
        in_specs=[pl.BlockSpec((2, B, 16), lambda i: (0, i, 0)),
                  pl.BlockSpec((B, F1), lambda i: (i, 0)),
                  pl.BlockSpec((B, F1), lambda i: (i, 0)),
                  _full_spec((F1, F1))],
        out_specs=_row2_spec(F1),
        out_shape=jax.ShapeDtypeStruct((2, N, F1), jnp.bfloat16),
    )(degw, x1, x2, W1)


def _layer_a_body(d, dnext, degw_ref, acc_ref, hs_ref, b_ref, br_ref,
                  aw1_ref, ab1_ref, aw2_ref, ab2_ref, wn_ref,
                  f_ref, sseg_ref, cnt_ref, hsn_ref):
    i = pl.program_id(0)
    iota_gb = lax.broadcasted_iota(jnp.int32, (G, B), 0)

    @pl.when(i == 0)
    def _():
        sseg_ref[...] = jnp.zeros_like(sseg_ref)
        cnt_ref[...] = jnp.zeros_like(cnt_ref)

    for g in range(2):
        dinv_d = _dot(lax.rsqrt(degw_ref[g] + 1.0), _col0_to(d))
        f = jnp.maximum(
            dinv_d * (acc_ref[g].astype(jnp.float32) +
                      hs_ref[g].astype(jnp.float32)) + b_ref[...], 0.0)
        f_ref[g] = f
        u = jnp.maximum(_dot(f, aw1_ref[...]) + ab1_ref[...], 0.0)
        att = jnp.tanh(_dot(u, aw2_ref[...]) + ab2_ref[...])
        Pt = (jnp.broadcast_to(br_ref[g, 0], (G, B)) == iota_gb
              ).astype(jnp.float32)
        sseg_ref[g] += _dot(Pt, f * att)
        cnt_ref[g] += _dot(Pt, jnp.ones((B, 1), jnp.float32))
        if dnext:
            dinv_dn = _dot(lax.rsqrt(degw_ref[g] + 1.0), _col0_to(dnext))
            hsn_ref[g] = (dinv_dn * _dot(f, wn_ref[...])
                          ).astype(jnp.bfloat16)


@functools.partial(jax.jit, static_argnames=("d", "dnext"))
def _tc_layer_a(degw, acc, hs, b, br3, aw1, ab1, aw2, ab2, wn, d, dnext):
    r = aw1.shape[1]
    dn = dnext or 8
    return pl.pallas_call(
        functools.partial(_layer_a_body, d, dnext),
        grid=(NB,),
        in_specs=[pl.BlockSpec((2, B, 16), lambda i: (0, i, 0)),
                  _row2_spec(d), _row2_spec(d),
                  _full_spec((1, d)),
                  pl.BlockSpec((2, 1, 1, B), lambda i: (0, i, 0, 0)),
                  _full_spec((d, r)), _full_spec((1, r)),
                  _full_spec((r, d)), _full_spec((1, d)),
                  _full_spec((d, dn))],
        out_specs=[_row2_spec(d), _full_spec((2, G, d)),
                   _full_spec((2, G, 1)), _row2_spec(dn)],
        out_shape=[jax.ShapeDtypeStruct((2, N, d), jnp.float32),
                   jax.ShapeDtypeStruct((2, G, d), jnp.float32),
                   jax.ShapeDtypeStruct((2, G, 1), jnp.float32),
                   jax.ShapeDtypeStruct((2, N, dn), jnp.bfloat16)],
    )(degw, acc, hs, b, br3, aw1, ab1, aw2, ab2, wn)


def _ntn(e1, e2, tW, tWbT, tb, d, dh):
    T3 = _dot(e1, tW)
    scoring = jnp.sum(T3.reshape(G, d, dh) * e2[:, :, None], axis=1)
    block = _dot(jnp.concatenate([e1, e2], axis=1), tWbT)
    return jnp.maximum(scoring + block + tb, 0.0)


def _layer_b_body(d, final, *refs):
    if final:
        (f_ref, bc_ref, br_ref, sseg_ref, cnt_ref,
         tW_ref, tWbT_ref, tb_ref, s1_ref, s2_ref,
         sew1_ref, seb1_ref, sew2_ref, seb2_ref, fcw_ref, fcb_ref,
         e_ref, out_ref) = refs
    else:
        (f_ref, bc_ref, br_ref, sseg_ref, cnt_ref,
         tW_ref, tWbT_ref, tb_ref,
         e_ref, out_ref) = refs
    i = pl.program_id(0)
    iota_bg = lax.broadcasted_iota(jnp.int32, (B, G), 1)
    iota_gb = lax.broadcasted_iota(jnp.int32, (G, B), 0)

    @pl.when(i == 0)
    def _():
        e_ref[...] = jnp.zeros_like(e_ref)

    for g in range(2):
        f = f_ref[g]
        tg = jnp.tanh(sseg_ref[g] / jnp.maximum(cnt_ref[g], 1.0))
        P = (jnp.broadcast_to(bc_ref[g, 0], (B, G)) == iota_bg
             ).astype(jnp.float32)
        Pt = (jnp.broadcast_to(br_ref[g, 0], (G, B)) == iota_gb
              ).astype(jnp.float32)
        tgn = _dot(P, tg)
        coefs_d = jax.nn.sigmoid(
            _dot(f * tgn, jnp.ones((d, d), jnp.float32)) * 10.0)
        e_ref[g] += _dot(Pt, coefs_d * f)

    @pl.when(i == NB - 1)
    def _():
        s = _ntn(e_ref[0], e_ref[1], tW_ref[...], tWbT_ref[...],
                 tb_ref[...], d, d // 2)
        if final:
            scores = jnp.concatenate([s, s2_ref[...], s1_ref[...]], axis=1)
            se = jax.nn.sigmoid(
                _dot(jnp.maximum(_dot(scores, sew1_ref[...]) + seb1_ref[...],
                                 0.0), sew2_ref[...]) + seb2_ref[...])
            out_ref[...] = jnp.maximum(
                _dot(se * scores + scores, fcw_ref[...]) + fcb_ref[...], 0.0)
        else:
            out_ref[...] = s


@functools.partial(jax.jit, static_argnames=("d", "final"))
def _tc_layer_b(f, bc3, br3, sseg, cnt, tWf, tWbT, tb, extras, d, final):
    dh = d // 2
    in_specs = [_row2_spec(d),
                pl.BlockSpec((2, 1, B, 1), lambda i: (0, i, 0, 0)),
                pl.BlockSpec((2, 1, 1, B), lambda i: (0, i, 0, 0)),
                _full_spec((2, G, d)), _full_spec((2, G, 1)),
                _full_spec((d, d * dh)), _full_spec((2 * d, dh)),
                _full_spec((1, dh))]
    args = [f, bc3, br3, sseg, cnt, tWf, tWbT, tb]
    if final:
        in_specs += [_full_spec(x.shape) for x in extras]
        args += list(extras)
    out_specs = [_full_spec((2, G, d)),
                 _full_spec((G, 64 if final else dh))]
    out_shape = [jax.ShapeDtypeStruct((2, G, d), jnp.float32),
                 jax.ShapeDtypeStruct((G, 64 if final else dh),
                                      jnp.float32)]
    return pl.pallas_call(
        functools.partial(_layer_b_body, d, final),
        grid=(NB,),
        in_specs=in_specs,
        out_specs=out_specs,
        out_shape=out_shape,
    )(*args)


def kernel(edge_index_1, features_1, batch_1, i_1, edge_index_2, features_2,
           batch_2, i_2, W1, b1, W2, b2, W3, b3, a1w1, a1b1, a1w2, a1b2,
           a2w1, a2b1, a2w2, a2b2, a3w1, a3b1, a3w2, a3b2, t1W, t1Wb, t1b,
           t2W, t2Wb, t2b, t3W, t3Wb, t3b, fc_w, fc_b, se_w1, se_b1,
           se_w2, se_b2):
    src_flat = jnp.concatenate([edge_index_1[0], edge_index_2[0] + N])
    dst_flat = jnp.concatenate([edge_index_1[1], edge_index_2[1]])
    batch = jnp.stack([batch_1, batch_2])
    bc3 = batch.reshape(2, NB, B, 1)
    br3 = batch.reshape(2, NB, 1, B)
    tWf = (t1W.reshape(F1, F1 * (F1 // 2)),
           t2W.reshape(F2, F2 * (F2 // 2)),
           t3W.reshape(F3, F3 * (F3 // 2)))
    tWbT = (t1Wb.T, t2Wb.T, t3Wb.T)
    tb = (t1b.reshape(1, -1), t2b.reshape(1, -1), t3b.reshape(1, -1))
    aws = ((a1w1, a1b1.reshape(1, -1), a1w2, a1b2.reshape(1, -1)),
           (a2w1, a2b1.reshape(1, -1), a2w2, a2b2.reshape(1, -1)),
           (a3w1, a3b1.reshape(1, -1), a3w2, a3b2.reshape(1, -1)))
    bs = (b1.reshape(1, F1), b2.reshape(1, F2), b3.reshape(1, F3))
    Wn = (W2, W3, None)
    dims = (F1, F2, F3)
    ones_rows = jnp.ones((ECH, 16), jnp.float32)
    z16 = jnp.zeros((RPT, 16), jnp.float32)
    zd = {dd: jnp.zeros((RPT, dd), jnp.bfloat16) for dd in dims}

    degw = _sc_degree(dst_flat, ones_rows, z16)
    hs = _tc_prep(degw, features_1, features_2, W1)

    ss = []
    out = None
    for l in range(3):
        d, dnext = dims[l], (dims[l + 1] if l < 2 else 0)
        acc = _sc_edge_pass(hs.reshape(2 * N, d), src_flat, dst_flat,
                            zd[d], d=d, kdepth=2 if d == F1 else 4)
        aw1, ab1, aw2, ab2 = aws[l]
        wn = Wn[l] if Wn[l] is not None else jnp.zeros((d, 8), jnp.float32)
        f, sseg, cnt, hsn = _tc_layer_a(degw, acc, hs, bs[l], br3,
                                        aw1, ab1, aw2, ab2, wn,
                                        d=d, dnext=dnext)
        final = l == 2
        extras = ((ss[0], ss[1], se_w1, se_b1.reshape(1, -1), se_w2,
                   se_b2.reshape(1, -1), fc_w, fc_b.reshape(1, -1))
                  if final else ())
        e, out_l = _tc_layer_b(f, bc3, br3, sseg, cnt,
                               tWf[l], tWbT[l], tb[l], extras,
                               d=d, final=final)
        if final:
            out = out_l
        else:
            ss.append(out_l)
            hs = hsn
    return out
